# baseline (speedup 1.0000x reference)
import functools

import jax
import jax.numpy as jnp
from jax import lax
from jax.experimental import pallas as pl
from jax.experimental.pallas import tpu as pltpu
from jax.experimental.pallas import tpu_sc as plsc

_NT = 5
_NG = 256
_N = 2000
_E = 6400
_H = 128

_NSUB = 16
_EPP = 512
_EP = _EPP * _NSUB
_NP = 2048
_RPS = _NP // _NSUB
_EPC = 13


def _lrelu(x, slope):
    return jnp.where(x >= 0, x, x * slope)


def _stage_a_body(x_ref, W_ref, b_ref, Wgs_ref, asrc_ref, Wgd_ref, adst_ref,
                  eaT_ref, We_ref, ae_ref,
                  xn_ref, S_ref, aeg_ref):
    xn = _lrelu(jnp.dot(x_ref[0], W_ref[0], preferred_element_type=jnp.float32)
                + b_ref[0], 0.01)
    xn_ref[0] = xn
    cols = []
    for dt in range(_NT):
        row = asrc_ref[0, dt][None, :]
        cols.append(jnp.sum(Wgs_ref[0, dt] * row, axis=1, keepdims=True))
        erow = ae_ref[0, dt][None, :]
        we = jnp.sum(We_ref[0, dt] * erow, axis=1, keepdims=True)
        aeg_ref[0, dt] = jnp.sum(eaT_ref[0, dt] * we, axis=0, keepdims=True)
    for st in range(_NT):
        drow = adst_ref[st, 0]
        cols.append(jnp.sum(Wgd_ref[st, 0] * drow, axis=1, keepdims=True))
    cols.append(jnp.zeros((_H, _H - 2 * _NT), jnp.float32))
    M = jnp.concatenate(cols, axis=1)
    S_ref[0] = jnp.dot(xn, M, preferred_element_type=jnp.float32)


def _stage_a(x, W, b, Wg_r, asrc_r, adst_r, eaT_r, We_r, ae_r):
    din = x.shape[-1]
    f32 = jnp.float32
    return pl.pallas_call(
        _stage_a_body,
        grid=(_NT,),
        in_specs=[
            pl.BlockSpec((1, _N, din), lambda t: (t, 0, 0)),
            pl.BlockSpec((1, din, _H), lambda t: (t, 0, 0)),
            pl.BlockSpec((1, 1, _H), lambda t: (t, 0, 0)),
            pl.BlockSpec((1, _NT, _H, _H), lambda t: (t, 0, 0, 0)),
            pl.BlockSpec((1, _NT, _H), lambda t: (t, 0, 0)),
            pl.BlockSpec((_NT, 1, _H, _H), lambda t: (0, t, 0, 0)),
            pl.BlockSpec((_NT, 1, 1, _H), lambda t: (0, t, 0, 0)),
            pl.BlockSpec((1, _NT, 4, _E), lambda t: (t, 0, 0, 0)),
            pl.BlockSpec((1, _NT, 4, _H), lambda t: (t, 0, 0, 0)),
            pl.BlockSpec((1, _NT, _H), lambda t: (t, 0, 0)),
        ],
        out_specs=[
            pl.BlockSpec((1, _N, _H), lambda t: (t, 0, 0)),
            pl.BlockSpec((1, _N, _H), lambda t: (t, 0, 0)),
            pl.BlockSpec((1, _NT, 1, _E), lambda t: (t, 0, 0, 0)),
        ],
        out_shape=[
            jax.ShapeDtypeStruct((_NT, _N, _H), f32),
            jax.ShapeDtypeStruct((_NT, _N, _H), f32),
            jax.ShapeDtypeStruct((_NT, _NT, 1, _E), f32),
        ],
    )(x, W, b.reshape(_NT, 1, _H), Wg_r, asrc_r, Wg_r,
      adst_r.reshape(_NT, _NT, 1, _H), eaT_r, We_r, ae_r)


def _sc_edge_body(xflat, ssrc, sdst, aeg, esrc, edst, agg_out,
                  s_src_l, s_dst_l, aeg_l, src_l, dst_l, src2d, dst2d,
                  ex_l, a_l, den_l, rows, zbuf, z816, iota_r, den_sh,
                  agg_sh, sem_in, sem_z, sem_g, sem_s):
    c = lax.axis_index("c")
    s = lax.axis_index("s")
    base = s * _EPP
    zf = jnp.zeros((16,), jnp.float32)

    def _zb(i, carry):
        r = i // 8
        k = i % 8
        zbuf[r, pl.ds(k * 16, 16)] = zf
        return carry
    lax.fori_loop(0, _RPS * 8, _zb, 0)
    for r in range(8):
        z816[r, :] = zf
    for p2 in range(2):
        for k in range(8):
            iota_r[p2, pl.ds(k * 16, 16)] = (lax.iota(jnp.int32, 16)
                                             + k * 16 + p2 * 128)

    pltpu.sync_copy(zbuf, agg_sh.at[pl.ds(s * _RPS, _RPS)])
    pltpu.sync_copy(z816, den_sh.at[pl.ds(s * 8, 8)])
    plsc.subcore_barrier()

    def _per_edge_type(i, carry):
        e = jnp.minimum(c * _EPC + i, 24)
        st = e // _NT
        p = i % 2
        q = 1 - p

        d_in = [
            pltpu.async_copy(ssrc.at[e], s_src_l, sem_in),
            pltpu.async_copy(sdst.at[e], s_dst_l, sem_in),
            pltpu.async_copy(aeg.at[e].at[pl.ds(base, _EPP)], aeg_l, sem_in),
            pltpu.async_copy(esrc.at[e].at[pl.ds(base, _EPP)], src_l, sem_in),
            pltpu.async_copy(edst.at[e].at[pl.ds(base, _EPP)], dst_l, sem_in),
        ]

        @pl.when(i > 0)
        def _():
            e_prev = jnp.minimum(c * _EPC + i - 1, 24)
            pltpu.sync_copy(agg_sh.at[pl.ds(s * _RPS, _RPS)],
                            agg_out.at[e_prev].at[pl.ds(s * _RPS, _RPS)])
        d_z = [
            pltpu.async_copy(zbuf, agg_sh.at[pl.ds(s * _RPS, _RPS)], sem_z),
            pltpu.async_copy(z816, den_sh.at[pl.ds(q * 128 + s * 8, 8)], sem_z),
        ]

        def _zd(r, carry2):
            den_l[r] = zf
            return carry2
        lax.fori_loop(0, 128, _zd, 0)
        for d in d_in:
            d.wait()

        def _idx(g, carry2):
            o = g * 16
            src2d[g // 8, pl.ds((g % 8) * 16, 16)] = src_l[pl.ds(o, 16)] + st * _N
            dst2d[g // 8, pl.ds((g % 8) * 16, 16)] = dst_l[pl.ds(o, 16)]
            return carry2
        lax.fori_loop(0, _EPP // 16, _idx, 0)
        d_g = [pltpu.async_copy(xflat.at[src2d.at[j]],
                                rows.at[pl.ds(j * 128, 128)], sem_g)
               for j in range(_EPP // 128)]

        def _p1(g, carry2):
            o = g * 16
            vs = src_l[pl.ds(o, 16)]
            vd = dst_l[pl.ds(o, 16)]
            a1 = plsc.load_gather(s_src_l, [vs])
            a2 = plsc.load_gather(s_dst_l, [jnp.minimum(vd, _N - 1)])
            al = a1 + a2 + aeg_l[pl.ds(o, 16)]
            al = jnp.where(al >= 0, al, al * 0.2)
            ex = jnp.exp(al)
            ex_l[pl.ds(o, 16)] = ex
            plsc.addupdate_scatter(den_l, [vd // 16, vd % 16], ex)
            return carry2
        lax.fori_loop(0, _EPP // 16, _p1, 0)

        pltpu.sync_copy(den_l, den_sh.at[iota_r.at[p]], add=True)
        for d in d_z:
            d.wait()
        plsc.subcore_barrier()
        pltpu.sync_copy(den_sh.at[pl.ds(p * 128, 128)], den_l)

        def _p2(g, carry2):
            o = g * 16
            vd = dst_l[pl.ds(o, 16)]
            dv = plsc.load_gather(den_l, [vd // 16, vd % 16])
            a_l[pl.ds(o, 16)] = ex_l[pl.ds(o, 16)] / (dv + 1e-16)
            return carry2
        lax.fori_loop(0, _EPP // 16, _p2, 0)

        for d in d_g:
            d.wait()

        def _scale(i2, carry2):
            ab = plsc.load_gather(a_l, [jnp.full((16,), 0, jnp.int32) + i2])
            for k in range(8):
                rows[i2, pl.ds(k * 16, 16)] = rows[i2, pl.ds(k * 16, 16)] * ab
            return carry2
        lax.fori_loop(0, _EPP, _scale, 0)

        d_s = [pltpu.async_copy(rows.at[pl.ds(j * 128, 128)],
                                agg_sh.at[dst2d.at[j]], sem_s, add=True)
               for j in range(_EPP // 128)]
        for d in d_s:
            d.wait()
        plsc.subcore_barrier()
        return carry

    lax.fori_loop(0, _EPC, _per_edge_type, 0)

    e_last = jnp.minimum(c * _EPC + _EPC - 1, 24)
    pltpu.sync_copy(agg_sh.at[pl.ds(s * _RPS, _RPS)],
                    agg_out.at[e_last].at[pl.ds(s * _RPS, _RPS)])


@functools.cache
def _make_sc_edge_aggregate():
    @functools.partial(
        pl.kernel,
        mesh=plsc.VectorSubcoreMesh(core_axis_name="c", subcore_axis_name="s"),
        out_type=jax.ShapeDtypeStruct((_NT * _NT, _NP, _H), jnp.float32),
        compiler_params=pltpu.CompilerParams(needs_layout_passes=False),
        scratch_types=[
            pltpu.VMEM((_N,), jnp.float32),
            pltpu.VMEM((_N,), jnp.float32),
            pltpu.VMEM((_EPP,), jnp.float32),
            pltpu.VMEM((_EPP,), jnp.int32),
            pltpu.VMEM((_EPP,), jnp.int32),
            pltpu.VMEM((4, 128), jnp.int32),
            pltpu.VMEM((4, 128), jnp.int32),
            pltpu.VMEM((_EPP,), jnp.float32),
            pltpu.VMEM((_EPP,), jnp.float32),
            pltpu.VMEM((128, 16), jnp.float32),
            pltpu.VMEM((_EPP, _H), jnp.float32),
            pltpu.VMEM((_RPS, _H), jnp.float32),
            pltpu.VMEM((8, 16), jnp.float32),
            pltpu.VMEM((2, 128), jnp.int32),
            pltpu.VMEM_SHARED((256, 16), jnp.float32),
            pltpu.VMEM_SHARED((_NP, _H), jnp.float32),
            pltpu.SemaphoreType.DMA,
            pltpu.SemaphoreType.DMA,
            pltpu.SemaphoreType.DMA,
            pltpu.SemaphoreType.DMA,
        ],
    )
    def _sc_edge_aggregate(xflat, ssrc, sdst, aeg, esrc, edst, agg_out, *rest):
        _sc_edge_body(xflat, ssrc, sdst, aeg, esrc, edst, agg_out, *rest)

    return _sc_edge_aggregate


def _stage_c_body(agg_ref, Wg_ref, bg_ref, x_ref, Wxc_ref, bxc_ref,
                  Wcc_ref, bcc_ref, Wat_ref, bat_ref, out_ref):
    comms = None
    for st in range(_NT):
        o = jnp.dot(agg_ref[st, 0], Wg_ref[st, 0],
                    preferred_element_type=jnp.float32) + bg_ref[st, 0]
        comms = o if comms is None else jnp.maximum(comms, o)
    cc = _lrelu(comms, 0.01)
    xv = x_ref[0]
    left = jnp.dot(xv, Wxc_ref[0], preferred_element_type=jnp.float32) + bxc_ref[0]
    right = jnp.dot(cc, Wcc_ref[0], preferred_element_type=jnp.float32) + bcc_ref[0]
    xt = jnp.concatenate([left, right], axis=1)
    att = jnp.dot(xt, Wat_ref[0], preferred_element_type=jnp.float32) + bat_ref[0]
    out_ref[0] = xt + jax.nn.sigmoid(att) * xt


def _stage_c(agg_r, Wg_r, bg_r, xn, Wxc, bxc, Wcc, bcc, Wat, bat):
    f32 = jnp.float32
    return pl.pallas_call(
        _stage_c_body,
        grid=(_NT,),
        in_specs=[
            pl.BlockSpec((_NT, 1, _N, _H), lambda t: (0, t, 0, 0)),
            pl.BlockSpec((_NT, 1, _H, _H), lambda t: (0, t, 0, 0)),
            pl.BlockSpec((_NT, 1, 1, _H), lambda t: (0, t, 0, 0)),
            pl.BlockSpec((1, _N, _H), lambda t: (t, 0, 0)),
            pl.BlockSpec((1, _H, _H // 2), lambda t: (t, 0, 0)),
            pl.BlockSpec((1, 1, _H // 2), lambda t: (t, 0, 0)),
            pl.BlockSpec((1, _H, _H // 2), lambda t: (t, 0, 0)),
            pl.BlockSpec((1, 1, _H // 2), lambda t: (t, 0, 0)),
            pl.BlockSpec((1, _H, _H), lambda t: (t, 0, 0)),
            pl.BlockSpec((1, 1, _H), lambda t: (t, 0, 0)),
        ],
        out_specs=pl.BlockSpec((1, _N, _H), lambda t: (t, 0, 0)),
        out_shape=jax.ShapeDtypeStruct((_NT, _N, _H), f32),
    )(agg_r, Wg_r, bg_r.reshape(_NT, _NT, 1, _H), xn,
      Wxc, bxc.reshape(_NT, 1, _H // 2), Wcc, bcc.reshape(_NT, 1, _H // 2),
      Wat, bat.reshape(_NT, 1, _H))


def _stage_d_body(x_ref, W2_ref, b2_ref, W3_ref, b3_ref, bb_ref,
                  wout_ref, bout_ref, out_ref):
    b2 = b2_ref[...]
    b3 = b3_ref[...]
    ssum = jnp.zeros((_NG, _H), jnp.float32)
    cnt = jnp.zeros((_NG,), jnp.float32)
    for t in range(_NT):
        y = _lrelu(jnp.dot(x_ref[t], W2_ref[t], preferred_element_type=jnp.float32)
                   + b2[t:t + 1], 0.01)
        y = _lrelu(jnp.dot(y, W3_ref[t], preferred_element_type=jnp.float32)
                   + b3[t:t + 1], 0.01)
        bb = bb_ref[t, :]
        iota = lax.broadcasted_iota(jnp.int32, (_N, _NG), 1)
        oh = (iota == bb[:, None]).astype(jnp.float32)
        ssum = ssum + lax.dot_general(oh, y, (((0,), (0,)), ((), ())),
                                      preferred_element_type=jnp.float32)
        cnt = cnt + oh.sum(0)
    pooled = ssum / jnp.maximum(cnt, 1.0)[:, None]
    out_ref[...] = jax.nn.sigmoid(
        jnp.dot(pooled, wout_ref[...], preferred_element_type=jnp.float32)
        + bout_ref[0])


def _stage_d(xs, W2, b2, W3, b3, batch_ids, W_out, b_out):
    return pl.pallas_call(
        _stage_d_body,
        out_shape=jax.ShapeDtypeStruct((_NG, 1), jnp.float32),
    )(xs, W2, b2, W3, b3, batch_ids, W_out, b_out)


def kernel(x, edge_index, edge_attr, batch_ids, W_sl0, b_sl0, W_sl, b_sl,
           W_gat, a_src, a_dst, W_edge, a_edge, b_gat, W_xc, b_xc, W_cc,
           b_cc, W_at, b_at, W_out, b_out):
    def _pad_edges(arr, cval):
        a3 = arr.reshape(_NT * _NT, _NSUB, _E // _NSUB)
        a3 = jnp.pad(a3, ((0, 0), (0, 0), (0, _EPP - _E // _NSUB)),
                     constant_values=cval)
        return a3.reshape(_NT * _NT, _EP)

    esrc = _pad_edges(edge_index[:, 0, :], 0)
    edst = _pad_edges(edge_index[:, 1, :], _N)
    eaT_r = edge_attr.transpose(0, 2, 1).reshape(_NT, _NT, 4, _E)

    xs = x
    for L in range(3):
        Wg_r = W_gat[L].reshape(_NT, _NT, _H, _H)
        asrc_r = a_src[L].reshape(_NT, _NT, _H)
        adst_r = a_dst[L].reshape(_NT, _NT, _H)
        We_r = W_edge[L].reshape(_NT, _NT, 4, _H)
        ae_r = a_edge[L].reshape(_NT, _NT, _H)
        if L == 0:
            Wl, bl = W_sl0, b_sl0
        else:
            Wl, bl = W_sl[L - 1], b_sl[L - 1]
        xn, S, aeg4 = _stage_a(xs, Wl, bl, Wg_r, asrc_r, adst_r,
                               eaT_r, We_r, ae_r)
        ssrc = S[:, :, :_NT].transpose(0, 2, 1).reshape(_NT * _NT, _N)
        sdst = S[:, :, _NT:2 * _NT].transpose(2, 0, 1).reshape(_NT * _NT, _N)
        aeg_p = _pad_edges(aeg4.reshape(_NT * _NT, _E), 0.0)
        agg = _make_sc_edge_aggregate()(
            xn.reshape(_NT * _N, _H), ssrc, sdst, aeg_p, esrc, edst)
        xs = _stage_c(agg[:, :_N, :].reshape(_NT, _NT, _N, _H), Wg_r,
                      b_gat[L].reshape(_NT, _NT, _H), xn,
                      W_xc[L], b_xc[L], W_cc[L], b_cc[L], W_at[L], b_at[L])
    return _stage_d(xs, W_sl[2], b_sl[2], W_sl[3], b_sl[3],
                    batch_ids, W_out, b_out)


# speedup vs baseline: 5.3196x; 5.3196x over previous
import functools

import jax
import jax.numpy as jnp
from jax import lax
from jax.experimental import pallas as pl
from jax.experimental.pallas import tpu as pltpu
from jax.experimental.pallas import tpu_sc as plsc

_NT = 5
_NG = 256
_N = 2000
_E = 6400
_H = 128

_NSUB = 16
_EPP = 512
_EPS = 400
_BSZ = 80
_NB = _EPS // _BSZ
_EP = _EPP * _NSUB
_NP = 2048
_RPS = _NP // _NSUB
_EPC = 13


def _lrelu(x, slope):
    return jnp.where(x >= 0, x, x * slope)


def _stage_a_body(x_ref, W_ref, b_ref, Wgs_ref, asrc_ref, Wgd_ref, adst_ref,
                  eaT_ref, We_ref, ae_ref,
                  xn_ref, S_ref, aeg_ref):
    xn = _lrelu(jnp.dot(x_ref[0], W_ref[0], preferred_element_type=jnp.float32)
                + b_ref[0], 0.01)
    xn_ref[0] = xn
    cols = []
    for dt in range(_NT):
        row = asrc_ref[0, dt][None, :]
        cols.append(jnp.sum(Wgs_ref[0, dt] * row, axis=1, keepdims=True))
        erow = ae_ref[0, dt][None, :]
        we = jnp.sum(We_ref[0, dt] * erow, axis=1, keepdims=True)
        aeg_ref[0, dt] = jnp.sum(eaT_ref[0, dt] * we, axis=0, keepdims=True)
    for st in range(_NT):
        drow = adst_ref[st, 0]
        cols.append(jnp.sum(Wgd_ref[st, 0] * drow, axis=1, keepdims=True))
    cols.append(jnp.zeros((_H, _H - 2 * _NT), jnp.float32))
    M = jnp.concatenate(cols, axis=1)
    S_ref[0] = jnp.dot(xn, M, preferred_element_type=jnp.float32)


def _stage_a(x, W, b, Wg_r, asrc_r, adst_r, eaT_r, We_r, ae_r):
    din = x.shape[-1]
    f32 = jnp.float32
    return pl.pallas_call(
        _stage_a_body,
        grid=(_NT,),
        in_specs=[
            pl.BlockSpec((1, _N, din), lambda t: (t, 0, 0)),
            pl.BlockSpec((1, din, _H), lambda t: (t, 0, 0)),
            pl.BlockSpec((1, 1, _H), lambda t: (t, 0, 0)),
            pl.BlockSpec((1, _NT, _H, _H), lambda t: (t, 0, 0, 0)),
            pl.BlockSpec((1, _NT, _H), lambda t: (t, 0, 0)),
            pl.BlockSpec((_NT, 1, _H, _H), lambda t: (0, t, 0, 0)),
            pl.BlockSpec((_NT, 1, 1, _H), lambda t: (0, t, 0, 0)),
            pl.BlockSpec((1, _NT, 4, _E), lambda t: (t, 0, 0, 0)),
            pl.BlockSpec((1, _NT, 4, _H), lambda t: (t, 0, 0, 0)),
            pl.BlockSpec((1, _NT, _H), lambda t: (t, 0, 0)),
        ],
        out_specs=[
            pl.BlockSpec((1, _N, _H), lambda t: (t, 0, 0)),
            pl.BlockSpec((1, _N, _H), lambda t: (t, 0, 0)),
            pl.BlockSpec((1, _NT, 1, _E), lambda t: (t, 0, 0, 0)),
        ],
        out_shape=[
            jax.ShapeDtypeStruct((_NT, _N, _H), f32),
            jax.ShapeDtypeStruct((_NT, _N, _H), f32),
            jax.ShapeDtypeStruct((_NT, _NT, 1, _E), f32),
        ],
    )(x, W, b.reshape(_NT, 1, _H), Wg_r, asrc_r, Wg_r,
      adst_r.reshape(_NT, _NT, 1, _H), eaT_r, We_r, ae_r)


def _sc_edge_body(xflat, ssrc, sdst, aeg, esrc, edst, agg_out,
                  s_src_l, s_dst_l, aeg_l, src_l, dst_l, src2d, dst2d,
                  ex_l, a_l, den_l, rows, zbuf, z816, iota_r, den_sh,
                  agg_sh, sem_in, sem_z, sem_g, sem_s):
    c = lax.axis_index("c")
    s = lax.axis_index("s")
    base = s * _EPP
    zf = jnp.zeros((16,), jnp.float32)

    def _zb(i, carry):
        r = i // 8
        k = i % 8
        zbuf[r, pl.ds(k * 16, 16)] = zf
        return carry
    lax.fori_loop(0, _RPS * 8, _zb, 0)
    for r in range(8):
        z816[r, :] = zf
    for p2 in range(2):
        for k in range(8):
            iota_r[p2, pl.ds(k * 16, 16)] = (lax.iota(jnp.int32, 16)
                                             + k * 16 + p2 * 128)

    pltpu.sync_copy(zbuf, agg_sh.at[pl.ds(s * _RPS, _RPS)])
    pltpu.sync_copy(z816, den_sh.at[pl.ds(s * 8, 8)])
    plsc.subcore_barrier()

    def _per_edge_type(i, carry):
        e = jnp.minimum(c * _EPC + i, 24)
        st = e // _NT
        p = i % 2
        q = 1 - p

        d_in = [
            pltpu.async_copy(ssrc.at[e], s_src_l, sem_in),
            pltpu.async_copy(sdst.at[e], s_dst_l, sem_in),
            pltpu.async_copy(aeg.at[e].at[pl.ds(base, _EPP)], aeg_l, sem_in),
            pltpu.async_copy(esrc.at[e].at[pl.ds(base, _EPP)], src_l, sem_in),
            pltpu.async_copy(edst.at[e].at[pl.ds(base, _EPP)], dst_l, sem_in),
        ]

        @pl.when(i > 0)
        def _():
            e_prev = jnp.minimum(c * _EPC + i - 1, 24)
            pltpu.sync_copy(agg_sh.at[pl.ds(s * _RPS, _RPS)],
                            agg_out.at[e_prev].at[pl.ds(s * _RPS, _RPS)])
        d_z = [
            pltpu.async_copy(zbuf, agg_sh.at[pl.ds(s * _RPS, _RPS)], sem_z),
            pltpu.async_copy(z816, den_sh.at[pl.ds(q * 128 + s * 8, 8)], sem_z),
        ]

        def _zd(r, carry2):
            den_l[r] = zf
            return carry2
        lax.fori_loop(0, 128, _zd, 0)
        for d in d_in:
            d.wait()

        def _idx(g, carry2):
            o = g * 16
            src2d[g // 5, pl.ds((g % 5) * 16, 16)] = src_l[pl.ds(o, 16)] + st * _N
            dst2d[g // 5, pl.ds((g % 5) * 16, 16)] = dst_l[pl.ds(o, 16)]
            return carry2
        lax.fori_loop(0, _NB * 5, _idx, 0)
        d_g = [pltpu.async_copy(xflat.at[src2d.at[j]],
                                rows.at[pl.ds(j * _BSZ, _BSZ)], sem_g)
               for j in range(_NB)]

        def _p1(g, carry2):
            o = g * 16
            vs = src_l[pl.ds(o, 16)]
            vd = dst_l[pl.ds(o, 16)]
            a1 = plsc.load_gather(s_src_l, [vs])
            a2 = plsc.load_gather(s_dst_l, [vd])
            al = a1 + a2 + aeg_l[pl.ds(o, 16)]
            al = jnp.where(al >= 0, al, al * 0.2)
            ex = jnp.exp(al)
            ex_l[pl.ds(o, 16)] = ex
            plsc.addupdate_scatter(den_l, [vd // 16, vd % 16], ex)
            return carry2
        lax.fori_loop(0, _EPS // 16, _p1, 0)

        pltpu.sync_copy(den_l, den_sh.at[iota_r.at[p]], add=True)
        for d in d_z:
            d.wait()
        plsc.subcore_barrier()
        pltpu.sync_copy(den_sh.at[pl.ds(p * 128, 128)], den_l)

        def _p2(g, carry2):
            o = g * 16
            vd = dst_l[pl.ds(o, 16)]
            dv = plsc.load_gather(den_l, [vd // 16, vd % 16])
            a_l[pl.ds(o, 16)] = ex_l[pl.ds(o, 16)] / (dv + 1e-16)
            return carry2
        lax.fori_loop(0, _EPS // 16, _p2, 0)

        d_s = []
        for j in range(_NB):
            d_g[j].wait()

            def _scale(i2, carry2):
                ab = plsc.load_gather(a_l, [jnp.full((16,), 0, jnp.int32) + i2])
                for k in range(8):
                    rows[i2, pl.ds(k * 16, 16)] = rows[i2, pl.ds(k * 16, 16)] * ab
                return carry2
            lax.fori_loop(j * _BSZ, (j + 1) * _BSZ, _scale, 0)
            d_s.append(pltpu.async_copy(rows.at[pl.ds(j * _BSZ, _BSZ)],
                                        agg_sh.at[dst2d.at[j]], sem_s,
                                        add=True))
        for d in d_s:
            d.wait()
        plsc.subcore_barrier()
        return carry

    lax.fori_loop(0, _EPC, _per_edge_type, 0)

    e_last = jnp.minimum(c * _EPC + _EPC - 1, 24)
    pltpu.sync_copy(agg_sh.at[pl.ds(s * _RPS, _RPS)],
                    agg_out.at[e_last].at[pl.ds(s * _RPS, _RPS)])


@functools.cache
def _make_sc_edge_aggregate():
    @functools.partial(
        pl.kernel,
        mesh=plsc.VectorSubcoreMesh(core_axis_name="c", subcore_axis_name="s"),
        out_type=jax.ShapeDtypeStruct((_NT * _NT, _NP, _H), jnp.float32),
        compiler_params=pltpu.CompilerParams(needs_layout_passes=False),
        scratch_types=[
            pltpu.VMEM((_N,), jnp.float32),
            pltpu.VMEM((_N,), jnp.float32),
            pltpu.VMEM((_EPP,), jnp.float32),
            pltpu.VMEM((_EPP,), jnp.int32),
            pltpu.VMEM((_EPP,), jnp.int32),
            pltpu.VMEM((_NB, _BSZ), jnp.int32),
            pltpu.VMEM((_NB, _BSZ), jnp.int32),
            pltpu.VMEM((_EPP,), jnp.float32),
            pltpu.VMEM((_EPP,), jnp.float32),
            pltpu.VMEM((128, 16), jnp.float32),
            pltpu.VMEM((_EPP, _H), jnp.float32),
            pltpu.VMEM((_RPS, _H), jnp.float32),
            pltpu.VMEM((8, 16), jnp.float32),
            pltpu.VMEM((2, 128), jnp.int32),
            pltpu.VMEM_SHARED((256, 16), jnp.float32),
            pltpu.VMEM_SHARED((_NP, _H), jnp.float32),
            pltpu.SemaphoreType.DMA,
            pltpu.SemaphoreType.DMA,
            pltpu.SemaphoreType.DMA,
            pltpu.SemaphoreType.DMA,
        ],
    )
    def _sc_edge_aggregate(xflat, ssrc, sdst, aeg, esrc, edst, agg_out, *rest):
        _sc_edge_body(xflat, ssrc, sdst, aeg, esrc, edst, agg_out, *rest)

    return _sc_edge_aggregate


def _stage_c_body(agg_ref, Wg_ref, bg_ref, x_ref, Wxc_ref, bxc_ref,
                  Wcc_ref, bcc_ref, Wat_ref, bat_ref, out_ref):
    comms = None
    for st in range(_NT):
        o = jnp.dot(agg_ref[st, 0], Wg_ref[st, 0],
                    preferred_element_type=jnp.float32) + bg_ref[st, 0]
        comms = o if comms is None else jnp.maximum(comms, o)
    cc = _lrelu(comms, 0.01)
    xv = x_ref[0]
    left = jnp.dot(xv, Wxc_ref[0], preferred_element_type=jnp.float32) + bxc_ref[0]
    right = jnp.dot(cc, Wcc_ref[0], preferred_element_type=jnp.float32) + bcc_ref[0]
    xt = jnp.concatenate([left, right], axis=1)
    att = jnp.dot(xt, Wat_ref[0], preferred_element_type=jnp.float32) + bat_ref[0]
    out_ref[0] = xt + jax.nn.sigmoid(att) * xt


def _stage_c(agg_r, Wg_r, bg_r, xn, Wxc, bxc, Wcc, bcc, Wat, bat):
    f32 = jnp.float32
    return pl.pallas_call(
        _stage_c_body,
        grid=(_NT,),
        in_specs=[
            pl.BlockSpec((_NT, 1, _N, _H), lambda t: (0, t, 0, 0)),
            pl.BlockSpec((_NT, 1, _H, _H), lambda t: (0, t, 0, 0)),
            pl.BlockSpec((_NT, 1, 1, _H), lambda t: (0, t, 0, 0)),
            pl.BlockSpec((1, _N, _H), lambda t: (t, 0, 0)),
            pl.BlockSpec((1, _H, _H // 2), lambda t: (t, 0, 0)),
            pl.BlockSpec((1, 1, _H // 2), lambda t: (t, 0, 0)),
            pl.BlockSpec((1, _H, _H // 2), lambda t: (t, 0, 0)),
            pl.BlockSpec((1, 1, _H // 2), lambda t: (t, 0, 0)),
            pl.BlockSpec((1, _H, _H), lambda t: (t, 0, 0)),
            pl.BlockSpec((1, 1, _H), lambda t: (t, 0, 0)),
        ],
        out_specs=pl.BlockSpec((1, _N, _H), lambda t: (t, 0, 0)),
        out_shape=jax.ShapeDtypeStruct((_NT, _N, _H), f32),
    )(agg_r, Wg_r, bg_r.reshape(_NT, _NT, 1, _H), xn,
      Wxc, bxc.reshape(_NT, 1, _H // 2), Wcc, bcc.reshape(_NT, 1, _H // 2),
      Wat, bat.reshape(_NT, 1, _H))


def _stage_d_body(x_ref, W2_ref, b2_ref, W3_ref, b3_ref, bb_ref,
                  wout_ref, bout_ref, out_ref):
    b2 = b2_ref[...]
    b3 = b3_ref[...]
    ssum = jnp.zeros((_NG, _H), jnp.float32)
    cnt = jnp.zeros((_NG,), jnp.float32)
    for t in range(_NT):
        y = _lrelu(jnp.dot(x_ref[t], W2_ref[t], preferred_element_type=jnp.float32)
                   + b2[t:t + 1], 0.01)
        y = _lrelu(jnp.dot(y, W3_ref[t], preferred_element_type=jnp.float32)
                   + b3[t:t + 1], 0.01)
        bb = bb_ref[t, :]
        iota = lax.broadcasted_iota(jnp.int32, (_N, _NG), 1)
        oh = (iota == bb[:, None]).astype(jnp.float32)
        ssum = ssum + lax.dot_general(oh, y, (((0,), (0,)), ((), ())),
                                      preferred_element_type=jnp.float32)
        cnt = cnt + oh.sum(0)
    pooled = ssum / jnp.maximum(cnt, 1.0)[:, None]
    out_ref[...] = jax.nn.sigmoid(
        jnp.dot(pooled, wout_ref[...], preferred_element_type=jnp.float32)
        + bout_ref[0])


def _stage_d(xs, W2, b2, W3, b3, batch_ids, W_out, b_out):
    return pl.pallas_call(
        _stage_d_body,
        out_shape=jax.ShapeDtypeStruct((_NG, 1), jnp.float32),
    )(xs, W2, b2, W3, b3, batch_ids, W_out, b_out)


def kernel(x, edge_index, edge_attr, batch_ids, W_sl0, b_sl0, W_sl, b_sl,
           W_gat, a_src, a_dst, W_edge, a_edge, b_gat, W_xc, b_xc, W_cc,
           b_cc, W_at, b_at, W_out, b_out):
    def _pad_edges(arr, cval):
        a3 = arr.reshape(_NT * _NT, _NSUB, _E // _NSUB)
        a3 = jnp.pad(a3, ((0, 0), (0, 0), (0, _EPP - _E // _NSUB)),
                     constant_values=cval)
        return a3.reshape(_NT * _NT, _EP)

    esrc = _pad_edges(edge_index[:, 0, :], 0)
    edst = _pad_edges(edge_index[:, 1, :], _N)
    eaT_r = edge_attr.transpose(0, 2, 1).reshape(_NT, _NT, 4, _E)

    xs = x
    for L in range(3):
        Wg_r = W_gat[L].reshape(_NT, _NT, _H, _H)
        asrc_r = a_src[L].reshape(_NT, _NT, _H)
        adst_r = a_dst[L].reshape(_NT, _NT, _H)
        We_r = W_edge[L].reshape(_NT, _NT, 4, _H)
        ae_r = a_edge[L].reshape(_NT, _NT, _H)
        if L == 0:
            Wl, bl = W_sl0, b_sl0
        else:
            Wl, bl = W_sl[L - 1], b_sl[L - 1]
        xn, S, aeg4 = _stage_a(xs, Wl, bl, Wg_r, asrc_r, adst_r,
                               eaT_r, We_r, ae_r)
        ssrc = S[:, :, :_NT].transpose(0, 2, 1).reshape(_NT * _NT, _N)
        sdst = S[:, :, _NT:2 * _NT].transpose(2, 0, 1).reshape(_NT * _NT, _N)
        aeg_p = _pad_edges(aeg4.reshape(_NT * _NT, _E), 0.0)
        agg = _make_sc_edge_aggregate()(
            xn.reshape(_NT * _N, _H), ssrc, sdst, aeg_p, esrc, edst)
        xs = _stage_c(agg[:, :_N, :].reshape(_NT, _NT, _N, _H), Wg_r,
                      b_gat[L].reshape(_NT, _NT, _H), xn,
                      W_xc[L], b_xc[L], W_cc[L], b_cc[L], W_at[L], b_at[L])
    return _stage_d(xs, W_sl[2], b_sl[2], W_sl[3], b_sl[3],
                    batch_ids, W_out, b_out)


# speedup vs baseline: 6.0061x; 1.1290x over previous
import functools

import jax
import jax.numpy as jnp
from jax import lax
from jax.experimental import pallas as pl
from jax.experimental.pallas import tpu as pltpu
from jax.experimental.pallas import tpu_sc as plsc

_NT = 5
_NG = 256
_N = 2000
_E = 6400
_H = 128

_NSUB = 16
_EPP = 512
_EPS = 400
_BSZ = 80
_NB = _EPS // _BSZ
_EP = _EPP * _NSUB
_NP = 2048
_RPS = _NP // _NSUB
_EPC = 13


def _lrelu(x, slope):
    return jnp.where(x >= 0, x, x * slope)


def _stage_a_body(x_ref, W_ref, b_ref, Wgs_ref, asrc_ref, Wgd_ref, adst_ref,
                  eaT_ref, We_ref, ae_ref,
                  xn_ref, S_ref, aeg_ref):
    xn = _lrelu(jnp.dot(x_ref[0], W_ref[0], preferred_element_type=jnp.float32)
                + b_ref[0], 0.01)
    xn_ref[0] = xn
    cols = []
    for dt in range(_NT):
        row = asrc_ref[0, dt][None, :]
        cols.append(jnp.sum(Wgs_ref[0, dt] * row, axis=1, keepdims=True))
        erow = ae_ref[0, dt][None, :]
        we = jnp.sum(We_ref[0, dt] * erow, axis=1, keepdims=True)
        aeg_ref[0, dt] = jnp.sum(eaT_ref[0, dt] * we, axis=0, keepdims=True)
    for st in range(_NT):
        drow = adst_ref[st, 0]
        cols.append(jnp.sum(Wgd_ref[st, 0] * drow, axis=1, keepdims=True))
    cols.append(jnp.zeros((_H, _H - 2 * _NT), jnp.float32))
    M = jnp.concatenate(cols, axis=1)
    S_ref[0] = jnp.dot(xn, M, preferred_element_type=jnp.float32)


def _stage_a(x, W, b, Wg_r, asrc_r, adst_r, eaT_r, We_r, ae_r):
    din = x.shape[-1]
    f32 = jnp.float32
    return pl.pallas_call(
        _stage_a_body,
        grid=(_NT,),
        in_specs=[
            pl.BlockSpec((1, _N, din), lambda t: (t, 0, 0)),
            pl.BlockSpec((1, din, _H), lambda t: (t, 0, 0)),
            pl.BlockSpec((1, 1, _H), lambda t: (t, 0, 0)),
            pl.BlockSpec((1, _NT, _H, _H), lambda t: (t, 0, 0, 0)),
            pl.BlockSpec((1, _NT, _H), lambda t: (t, 0, 0)),
            pl.BlockSpec((_NT, 1, _H, _H), lambda t: (0, t, 0, 0)),
            pl.BlockSpec((_NT, 1, 1, _H), lambda t: (0, t, 0, 0)),
            pl.BlockSpec((1, _NT, 4, _E), lambda t: (t, 0, 0, 0)),
            pl.BlockSpec((1, _NT, 4, _H), lambda t: (t, 0, 0, 0)),
            pl.BlockSpec((1, _NT, _H), lambda t: (t, 0, 0)),
        ],
        out_specs=[
            pl.BlockSpec((1, _N, _H), lambda t: (t, 0, 0)),
            pl.BlockSpec((1, _N, _H), lambda t: (t, 0, 0)),
            pl.BlockSpec((1, _NT, 1, _E), lambda t: (t, 0, 0, 0)),
        ],
        out_shape=[
            jax.ShapeDtypeStruct((_NT, _N, _H), f32),
            jax.ShapeDtypeStruct((_NT, _N, _H), f32),
            jax.ShapeDtypeStruct((_NT, _NT, 1, _E), f32),
        ],
    )(x, W, b.reshape(_NT, 1, _H), Wg_r, asrc_r, Wg_r,
      adst_r.reshape(_NT, _NT, 1, _H), eaT_r, We_r, ae_r)


def _sc_edge_body(xflat, ssrc, sdst, aeg, esrc, edst, agg_out,
                  s_src_l, s_dst_l, aeg_l, src_l, dst_l, src2d, dst2d,
                  ex_l, a_l, den_l, rows, zbuf, z816, iota_r, den_sh,
                  agg_sh, sem_in, sem_z, sem_g, sem_s):
    c = lax.axis_index("c")
    s = lax.axis_index("s")
    base = s * _EPP
    zf = jnp.zeros((16,), jnp.float32)

    def _zb(i, carry):
        r = i // 8
        k = i % 8
        zbuf[r, pl.ds(k * 16, 16)] = zf
        return carry
    lax.fori_loop(0, _RPS * 8, _zb, 0)
    for r in range(8):
        z816[r, :] = zf
    for p2 in range(2):
        for k in range(8):
            iota_r[p2, pl.ds(k * 16, 16)] = (lax.iota(jnp.int32, 16)
                                             + k * 16 + p2 * 128)

    pltpu.sync_copy(zbuf, agg_sh.at[pl.ds(s * _RPS, _RPS)])
    pltpu.sync_copy(z816, den_sh.at[pl.ds(s * 8, 8)])
    plsc.subcore_barrier()

    def _per_edge_type(i, carry):
        e = jnp.minimum(c * _EPC + i, 24)
        st = e // _NT
        p = i % 2
        q = 1 - p

        d_in = [
            pltpu.async_copy(ssrc.at[e], s_src_l, sem_in),
            pltpu.async_copy(sdst.at[e], s_dst_l, sem_in),
            pltpu.async_copy(aeg.at[e].at[pl.ds(base, _EPP)], aeg_l, sem_in),
            pltpu.async_copy(esrc.at[e].at[pl.ds(base, _EPP)], src_l, sem_in),
            pltpu.async_copy(edst.at[e].at[pl.ds(base, _EPP)], dst_l, sem_in),
        ]

        @pl.when(i > 0)
        def _():
            e_prev = jnp.minimum(c * _EPC + i - 1, 24)
            pltpu.sync_copy(agg_sh.at[pl.ds(s * _RPS, _RPS)],
                            agg_out.at[e_prev].at[pl.ds(s * _RPS, _RPS)])
        d_z = [
            pltpu.async_copy(zbuf, agg_sh.at[pl.ds(s * _RPS, _RPS)], sem_z),
            pltpu.async_copy(z816, den_sh.at[pl.ds(q * 128 + s * 8, 8)], sem_z),
        ]

        def _zd(r, carry2):
            den_l[r] = zf
            return carry2
        lax.fori_loop(0, 128, _zd, 0)
        for d in d_in:
            d.wait()

        def _idx(g, carry2):
            o = g * 16
            src2d[g // 5, pl.ds((g % 5) * 16, 16)] = src_l[pl.ds(o, 16)] + st * _N
            dst2d[g // 5, pl.ds((g % 5) * 16, 16)] = dst_l[pl.ds(o, 16)]
            return carry2
        lax.fori_loop(0, _NB * 5, _idx, 0)
        d_g = [pltpu.async_copy(xflat.at[src2d.at[j]],
                                rows.at[pl.ds(j * _BSZ, _BSZ)], sem_g)
               for j in range(_NB)]

        def _p1(g, carry2):
            o = g * 16
            vs = src_l[pl.ds(o, 16)]
            vd = dst_l[pl.ds(o, 16)]
            a1 = plsc.load_gather(s_src_l, [vs])
            a2 = plsc.load_gather(s_dst_l, [vd])
            al = a1 + a2 + aeg_l[pl.ds(o, 16)]
            al = jnp.where(al >= 0, al, al * 0.2)
            ex = jnp.exp(al)
            ex_l[pl.ds(o, 16)] = ex
            plsc.addupdate_scatter(den_l, [vd // 16, vd % 16], ex)
            return carry2
        lax.fori_loop(0, _EPS // 16, _p1, 0)

        pltpu.sync_copy(den_l, den_sh.at[iota_r.at[p]], add=True)
        for d in d_z:
            d.wait()
        plsc.subcore_barrier()
        pltpu.sync_copy(den_sh.at[pl.ds(p * 128, 128)], den_l)

        def _p2(g, carry2):
            o = g * 16
            vd = dst_l[pl.ds(o, 16)]
            dv = plsc.load_gather(den_l, [vd // 16, vd % 16])
            a_l[pl.ds(o, 16)] = ex_l[pl.ds(o, 16)] / (dv + 1e-16)
            return carry2
        lax.fori_loop(0, _EPS // 16, _p2, 0)

        d_s = []
        for j in range(_NB):
            d_g[j].wait()

            def _scale(i2, carry2):
                ab = plsc.load_gather(a_l, [jnp.full((16,), 0, jnp.int32) + i2])
                for k in range(8):
                    rows[i2, pl.ds(k * 16, 16)] = rows[i2, pl.ds(k * 16, 16)] * ab
                return carry2
            lax.fori_loop(j * _BSZ, (j + 1) * _BSZ, _scale, 0)
            d_s.append(pltpu.async_copy(rows.at[pl.ds(j * _BSZ, _BSZ)],
                                        agg_sh.at[dst2d.at[j]], sem_s,
                                        add=True))
        for d in d_s:
            d.wait()
        plsc.subcore_barrier()
        return carry

    lax.fori_loop(0, _EPC, _per_edge_type, 0)

    e_last = jnp.minimum(c * _EPC + _EPC - 1, 24)
    pltpu.sync_copy(agg_sh.at[pl.ds(s * _RPS, _RPS)],
                    agg_out.at[e_last].at[pl.ds(s * _RPS, _RPS)])


@functools.cache
def _make_sc_edge_aggregate():
    @functools.partial(
        pl.kernel,
        mesh=plsc.VectorSubcoreMesh(core_axis_name="c", subcore_axis_name="s"),
        out_type=jax.ShapeDtypeStruct((_NT * _NT, _NP, _H), jnp.float32),
        compiler_params=pltpu.CompilerParams(needs_layout_passes=False),
        scratch_types=[
            pltpu.VMEM((_N,), jnp.float32),
            pltpu.VMEM((_N,), jnp.float32),
            pltpu.VMEM((_EPP,), jnp.float32),
            pltpu.VMEM((_EPP,), jnp.int32),
            pltpu.VMEM((_EPP,), jnp.int32),
            pltpu.VMEM((_NB, _BSZ), jnp.int32),
            pltpu.VMEM((_NB, _BSZ), jnp.int32),
            pltpu.VMEM((_EPP,), jnp.float32),
            pltpu.VMEM((_EPP,), jnp.float32),
            pltpu.VMEM((128, 16), jnp.float32),
            pltpu.VMEM((_EPP, _H), jnp.float32),
            pltpu.VMEM((_RPS, _H), jnp.float32),
            pltpu.VMEM((8, 16), jnp.float32),
            pltpu.VMEM((2, 128), jnp.int32),
            pltpu.VMEM_SHARED((256, 16), jnp.float32),
            pltpu.VMEM_SHARED((_NP, _H), jnp.float32),
            pltpu.SemaphoreType.DMA,
            pltpu.SemaphoreType.DMA,
            pltpu.SemaphoreType.DMA,
            pltpu.SemaphoreType.DMA,
        ],
    )
    def _sc_edge_aggregate(xflat, ssrc, sdst, aeg, esrc, edst, agg_out, *rest):
        _sc_edge_body(xflat, ssrc, sdst, aeg, esrc, edst, agg_out, *rest)

    return _sc_edge_aggregate


def _gate_block(agg_ref, Wg_ref, bg_ref, x_ref, Wxc_ref, bxc_ref,
                Wcc_ref, bcc_ref, Wat_ref, bat_ref):
    comms = None
    for st in range(_NT):
        o = jnp.dot(agg_ref[st, 0], Wg_ref[st, 0],
                    preferred_element_type=jnp.float32) + bg_ref[st, 0]
        comms = o if comms is None else jnp.maximum(comms, o)
    cc = _lrelu(comms, 0.01)
    xv = x_ref[0]
    left = jnp.dot(xv, Wxc_ref[0], preferred_element_type=jnp.float32) + bxc_ref[0]
    right = jnp.dot(cc, Wcc_ref[0], preferred_element_type=jnp.float32) + bcc_ref[0]
    xt = jnp.concatenate([left, right], axis=1)
    att = jnp.dot(xt, Wat_ref[0], preferred_element_type=jnp.float32) + bat_ref[0]
    return xt + jax.nn.sigmoid(att) * xt


def _a_block(x1, W_ref, b_ref, Wgs_ref, asrc_ref, Wgd_ref, adst_ref,
             eaT_ref, We_ref, ae_ref, xn_ref, S_ref, aeg_ref):
    xn = _lrelu(jnp.dot(x1, W_ref[0], preferred_element_type=jnp.float32)
                + b_ref[0], 0.01)
    xn_ref[0] = xn
    cols = []
    for dt in range(_NT):
        row = asrc_ref[0, dt][None, :]
        cols.append(jnp.sum(Wgs_ref[0, dt] * row, axis=1, keepdims=True))
        erow = ae_ref[0, dt][None, :]
        we = jnp.sum(We_ref[0, dt] * erow, axis=1, keepdims=True)
        aeg_ref[0, dt] = jnp.sum(eaT_ref[0, dt] * we, axis=0, keepdims=True)
    for st in range(_NT):
        drow = adst_ref[st, 0]
        cols.append(jnp.sum(Wgd_ref[st, 0] * drow, axis=1, keepdims=True))
    cols.append(jnp.zeros((_H, _H - 2 * _NT), jnp.float32))
    M = jnp.concatenate(cols, axis=1)
    S_ref[0] = jnp.dot(xn, M, preferred_element_type=jnp.float32)


def _stage_ca_body(agg_ref, Wg_ref, bg_ref, x_ref, Wxc_ref, bxc_ref,
                   Wcc_ref, bcc_ref, Wat_ref, bat_ref,
                   W_ref, b_ref, Wgs_ref, asrc_ref, Wgd_ref, adst_ref,
                   eaT_ref, We_ref, ae_ref,
                   xn_ref, S_ref, aeg_ref):
    x1 = _gate_block(agg_ref, Wg_ref, bg_ref, x_ref, Wxc_ref, bxc_ref,
                     Wcc_ref, bcc_ref, Wat_ref, bat_ref)
    _a_block(x1, W_ref, b_ref, Wgs_ref, asrc_ref, Wgd_ref, adst_ref,
             eaT_ref, We_ref, ae_ref, xn_ref, S_ref, aeg_ref)


def _stage_ca(agg_p, Wg_r, bg_r, xn, Wxc, bxc, Wcc, bcc, Wat, bat,
              W, b, Wg_n, asrc_n, adst_n, eaT_r, We_n, ae_n):
    f32 = jnp.float32
    return pl.pallas_call(
        _stage_ca_body,
        grid=(_NT,),
        in_specs=[
            pl.BlockSpec((_NT, 1, _N, _H), lambda t: (0, t, 0, 0)),
            pl.BlockSpec((_NT, 1, _H, _H), lambda t: (0, t, 0, 0)),
            pl.BlockSpec((_NT, 1, 1, _H), lambda t: (0, t, 0, 0)),
            pl.BlockSpec((1, _N, _H), lambda t: (t, 0, 0)),
            pl.BlockSpec((1, _H, _H // 2), lambda t: (t, 0, 0)),
            pl.BlockSpec((1, 1, _H // 2), lambda t: (t, 0, 0)),
            pl.BlockSpec((1, _H, _H // 2), lambda t: (t, 0, 0)),
            pl.BlockSpec((1, 1, _H // 2), lambda t: (t, 0, 0)),
            pl.BlockSpec((1, _H, _H), lambda t: (t, 0, 0)),
            pl.BlockSpec((1, 1, _H), lambda t: (t, 0, 0)),
            pl.BlockSpec((1, _H, _H), lambda t: (t, 0, 0)),
            pl.BlockSpec((1, 1, _H), lambda t: (t, 0, 0)),
            pl.BlockSpec((1, _NT, _H, _H), lambda t: (t, 0, 0, 0)),
            pl.BlockSpec((1, _NT, _H), lambda t: (t, 0, 0)),
            pl.BlockSpec((_NT, 1, _H, _H), lambda t: (0, t, 0, 0)),
            pl.BlockSpec((_NT, 1, 1, _H), lambda t: (0, t, 0, 0)),
            pl.BlockSpec((1, _NT, 4, _E), lambda t: (t, 0, 0, 0)),
            pl.BlockSpec((1, _NT, 4, _H), lambda t: (t, 0, 0, 0)),
            pl.BlockSpec((1, _NT, _H), lambda t: (t, 0, 0)),
        ],
        out_specs=[
            pl.BlockSpec((1, _N, _H), lambda t: (t, 0, 0)),
            pl.BlockSpec((1, _N, _H), lambda t: (t, 0, 0)),
            pl.BlockSpec((1, _NT, 1, _E), lambda t: (t, 0, 0, 0)),
        ],
        out_shape=[
            jax.ShapeDtypeStruct((_NT, _N, _H), f32),
            jax.ShapeDtypeStruct((_NT, _N, _H), f32),
            jax.ShapeDtypeStruct((_NT, _NT, 1, _E), f32),
        ],
    )(agg_p, Wg_r, bg_r.reshape(_NT, _NT, 1, _H), xn,
      Wxc, bxc.reshape(_NT, 1, _H // 2), Wcc, bcc.reshape(_NT, 1, _H // 2),
      Wat, bat.reshape(_NT, 1, _H),
      W, b.reshape(_NT, 1, _H), Wg_n, asrc_n, Wg_n,
      adst_n.reshape(_NT, _NT, 1, _H), eaT_r, We_n, ae_n)


def _stage_cd_body(agg_ref, Wg_ref, bg_ref, x_ref, Wxc_ref, bxc_ref,
                   Wcc_ref, bcc_ref, Wat_ref, bat_ref,
                   W2_ref, b2_ref, W3_ref, b3_ref, bb_ref, wout_ref,
                   bout_ref, out_ref, acc_ref, cnt_ref):
    t = pl.program_id(0)
    x1 = _gate_block(agg_ref, Wg_ref, bg_ref, x_ref, Wxc_ref, bxc_ref,
                     Wcc_ref, bcc_ref, Wat_ref, bat_ref)
    y = _lrelu(jnp.dot(x1, W2_ref[0], preferred_element_type=jnp.float32)
               + b2_ref[0], 0.01)
    y = _lrelu(jnp.dot(y, W3_ref[0], preferred_element_type=jnp.float32)
               + b3_ref[0], 0.01)
    bb = bb_ref[0, 0]
    iota = lax.broadcasted_iota(jnp.int32, (_N, _NG), 1)
    oh = (iota == bb[:, None]).astype(jnp.float32)
    part = lax.dot_general(oh, y, (((0,), (0,)), ((), ())),
                           preferred_element_type=jnp.float32)
    cpart = lax.dot_general(oh, jnp.ones((_N, 1), jnp.float32),
                            (((0,), (0,)), ((), ())),
                            preferred_element_type=jnp.float32)

    @pl.when(t == 0)
    def _():
        acc_ref[...] = part
        cnt_ref[...] = cpart

    @pl.when(t > 0)
    def _():
        acc_ref[...] += part
        cnt_ref[...] += cpart

    @pl.when(t == _NT - 1)
    def _():
        pooled = acc_ref[...] / jnp.maximum(cnt_ref[...], 1.0)
        out_ref[...] = jax.nn.sigmoid(
            jnp.dot(pooled, wout_ref[...], preferred_element_type=jnp.float32)
            + bout_ref[0])


def _stage_cd(agg_p, Wg_r, bg_r, xn, Wxc, bxc, Wcc, bcc, Wat, bat,
              W2, b2, W3, b3, batch_ids, W_out, b_out):
    f32 = jnp.float32
    return pl.pallas_call(
        _stage_cd_body,
        grid=(_NT,),
        in_specs=[
            pl.BlockSpec((_NT, 1, _N, _H), lambda t: (0, t, 0, 0)),
            pl.BlockSpec((_NT, 1, _H, _H), lambda t: (0, t, 0, 0)),
            pl.BlockSpec((_NT, 1, 1, _H), lambda t: (0, t, 0, 0)),
            pl.BlockSpec((1, _N, _H), lambda t: (t, 0, 0)),
            pl.BlockSpec((1, _H, _H // 2), lambda t: (t, 0, 0)),
            pl.BlockSpec((1, 1, _H // 2), lambda t: (t, 0, 0)),
            pl.BlockSpec((1, _H, _H // 2), lambda t: (t, 0, 0)),
            pl.BlockSpec((1, 1, _H // 2), lambda t: (t, 0, 0)),
            pl.BlockSpec((1, _H, _H), lambda t: (t, 0, 0)),
            pl.BlockSpec((1, 1, _H), lambda t: (t, 0, 0)),
            pl.BlockSpec((1, _H, _H), lambda t: (t, 0, 0)),
            pl.BlockSpec((1, 1, _H), lambda t: (t, 0, 0)),
            pl.BlockSpec((1, _H, _H), lambda t: (t, 0, 0)),
            pl.BlockSpec((1, 1, _H), lambda t: (t, 0, 0)),
            pl.BlockSpec((1, 1, _N), lambda t: (t, 0, 0)),
            pl.BlockSpec((_H, 1), lambda t: (0, 0)),
            pl.BlockSpec((1,), lambda t: (0,)),
        ],
        out_specs=pl.BlockSpec((_NG, 1), lambda t: (0, 0)),
        out_shape=jax.ShapeDtypeStruct((_NG, 1), f32),
        scratch_shapes=[
            pltpu.VMEM((_NG, _H), f32),
            pltpu.VMEM((_NG, 1), f32),
        ],
    )(agg_p, Wg_r, bg_r.reshape(_NT, _NT, 1, _H), xn,
      Wxc, bxc.reshape(_NT, 1, _H // 2), Wcc, bcc.reshape(_NT, 1, _H // 2),
      Wat, bat.reshape(_NT, 1, _H),
      W2, b2.reshape(_NT, 1, _H), W3, b3.reshape(_NT, 1, _H),
      batch_ids.reshape(_NT, 1, _N), W_out, b_out)


def kernel(x, edge_index, edge_attr, batch_ids, W_sl0, b_sl0, W_sl, b_sl,
           W_gat, a_src, a_dst, W_edge, a_edge, b_gat, W_xc, b_xc, W_cc,
           b_cc, W_at, b_at, W_out, b_out):
    def _pad_edges(arr, cval):
        a3 = arr.reshape(_NT * _NT, _NSUB, _E // _NSUB)
        a3 = jnp.pad(a3, ((0, 0), (0, 0), (0, _EPP - _E // _NSUB)),
                     constant_values=cval)
        return a3.reshape(_NT * _NT, _EP)

    esrc = _pad_edges(edge_index[:, 0, :], 0)
    edst = _pad_edges(edge_index[:, 1, :], _N)
    eaT_r = edge_attr.transpose(0, 2, 1).reshape(_NT, _NT, 4, _E)

    Wg_r = [W_gat[L].reshape(_NT, _NT, _H, _H) for L in range(3)]
    asrc_r = [a_src[L].reshape(_NT, _NT, _H) for L in range(3)]
    adst_r = [a_dst[L].reshape(_NT, _NT, _H) for L in range(3)]
    We_r = [W_edge[L].reshape(_NT, _NT, 4, _H) for L in range(3)]
    ae_r = [a_edge[L].reshape(_NT, _NT, _H) for L in range(3)]
    bg_r = [b_gat[L].reshape(_NT, _NT, _H) for L in range(3)]

    def _run_sc(xn, S, aeg4):
        ssrc = S[:, :, :_NT].transpose(0, 2, 1).reshape(_NT * _NT, _N)
        sdst = S[:, :, _NT:2 * _NT].transpose(2, 0, 1).reshape(_NT * _NT, _N)
        aeg_p = _pad_edges(aeg4.reshape(_NT * _NT, _E), 0.0)
        agg = _make_sc_edge_aggregate()(
            xn.reshape(_NT * _N, _H), ssrc, sdst, aeg_p, esrc, edst)
        return agg.reshape(_NT, _NT, _NP, _H)

    xn, S, aeg4 = _stage_a(x, W_sl0, b_sl0, Wg_r[0], asrc_r[0], adst_r[0],
                           eaT_r, We_r[0], ae_r[0])
    agg_p = _run_sc(xn, S, aeg4)
    for L in range(2):
        xn, S, aeg4 = _stage_ca(
            agg_p, Wg_r[L], bg_r[L], xn,
            W_xc[L], b_xc[L], W_cc[L], b_cc[L], W_at[L], b_at[L],
            W_sl[L], b_sl[L], Wg_r[L + 1], asrc_r[L + 1], adst_r[L + 1],
            eaT_r, We_r[L + 1], ae_r[L + 1])
        agg_p = _run_sc(xn, S, aeg4)
    return _stage_cd(agg_p, Wg_r[2], bg_r[2], xn,
                     W_xc[2], b_xc[2], W_cc[2], b_cc[2], W_at[2], b_at[2],
                     W_sl[2], b_sl[2], W_sl[3], b_sl[3],
                     batch_ids, W_out, b_out)


# speedup vs baseline: 6.7656x; 1.1265x over previous
import functools

import jax
import jax.numpy as jnp
from jax import lax
from jax.experimental import pallas as pl
from jax.experimental.pallas import tpu as pltpu
from jax.experimental.pallas import tpu_sc as plsc

_NT = 5
_NG = 256
_N = 2000
_E = 6400
_H = 128

_NSUB = 16
_EPP = 512
_EPS = 400
_BSZ = 80
_NB = _EPS // _BSZ
_EP = _EPP * _NSUB
_NP = 2048
_RPS = _NP // _NSUB
_EPC = 13


def _lrelu(x, slope):
    return jnp.where(x >= 0, x, x * slope)


def _stage_a_body(x_ref, W_ref, b_ref, Wgs_ref, asrc_ref, Wgd_ref, adst_ref,
                  eaT_ref, We_ref, ae_ref,
                  xn_ref, S_ref, aeg_ref):
    xn = _lrelu(jnp.dot(x_ref[0], W_ref[0], preferred_element_type=jnp.float32)
                + b_ref[0], 0.01)
    xn_ref[0] = xn
    cols = []
    for dt in range(_NT):
        row = asrc_ref[0, dt][None, :]
        cols.append(jnp.sum(Wgs_ref[0, dt] * row, axis=1, keepdims=True))
        erow = ae_ref[0, dt][None, :]
        we = jnp.sum(We_ref[0, dt] * erow, axis=1, keepdims=True)
        aeg_ref[0, dt] = jnp.sum(eaT_ref[0, dt] * we, axis=0, keepdims=True)
    for st in range(_NT):
        drow = adst_ref[st, 0]
        cols.append(jnp.sum(Wgd_ref[st, 0] * drow, axis=1, keepdims=True))
    cols.append(jnp.zeros((_H, _H - 2 * _NT), jnp.float32))
    M = jnp.concatenate(cols, axis=1)
    S_ref[0] = jnp.dot(xn, M, preferred_element_type=jnp.float32)


def _stage_a(x, W, b, Wg_r, asrc_r, adst_r, eaT_r, We_r, ae_r):
    din = x.shape[-1]
    f32 = jnp.float32
    return pl.pallas_call(
        _stage_a_body,
        grid=(_NT,),
        in_specs=[
            pl.BlockSpec((1, _N, din), lambda t: (t, 0, 0)),
            pl.BlockSpec((1, din, _H), lambda t: (t, 0, 0)),
            pl.BlockSpec((1, 1, _H), lambda t: (t, 0, 0)),
            pl.BlockSpec((1, _NT, _H, _H), lambda t: (t, 0, 0, 0)),
            pl.BlockSpec((1, _NT, _H), lambda t: (t, 0, 0)),
            pl.BlockSpec((_NT, 1, _H, _H), lambda t: (0, t, 0, 0)),
            pl.BlockSpec((_NT, 1, 1, _H), lambda t: (0, t, 0, 0)),
            pl.BlockSpec((1, _NT, 4, _E), lambda t: (t, 0, 0, 0)),
            pl.BlockSpec((1, _NT, 4, _H), lambda t: (t, 0, 0, 0)),
            pl.BlockSpec((1, _NT, _H), lambda t: (t, 0, 0)),
        ],
        out_specs=[
            pl.BlockSpec((1, _N, _H), lambda t: (t, 0, 0)),
            pl.BlockSpec((1, _N, _H), lambda t: (t, 0, 0)),
            pl.BlockSpec((1, _NT, 1, _E), lambda t: (t, 0, 0, 0)),
        ],
        out_shape=[
            jax.ShapeDtypeStruct((_NT, _N, _H), f32),
            jax.ShapeDtypeStruct((_NT, _N, _H), f32),
            jax.ShapeDtypeStruct((_NT, _NT, 1, _E), f32),
        ],
    )(x, W, b.reshape(_NT, 1, _H), Wg_r, asrc_r, Wg_r,
      adst_r.reshape(_NT, _NT, 1, _H), eaT_r, We_r, ae_r)


def _sc_edge_body(xflat, ssrc, sdst, aeg, esrc, edst, agg_out,
                  s_src_l, s_dst_l, aeg_l, src_l, dst_l, src2d, dst2d,
                  ex_l, a_l, den_l, rows, zbuf, z816, iota_r, den_sh,
                  agg_sh, sem_in, sem_z, sem_g, sem_s):
    c = lax.axis_index("c")
    s = lax.axis_index("s")
    base = s * _EPP
    zf = jnp.zeros((16,), jnp.float32)

    def _zb(i, carry):
        r = i // 8
        k = i % 8
        zbuf[r, pl.ds(k * 16, 16)] = zf
        return carry
    lax.fori_loop(0, _RPS * 8, _zb, 0)
    for r in range(8):
        z816[r, :] = zf
    for p2 in range(2):
        for k in range(8):
            iota_r[p2, pl.ds(k * 16, 16)] = (lax.iota(jnp.int32, 16)
                                             + k * 16 + p2 * 128)

    pltpu.sync_copy(zbuf, agg_sh.at[pl.ds(s * _RPS, _RPS)])
    pltpu.sync_copy(z816, den_sh.at[pl.ds(s * 8, 8)])
    plsc.subcore_barrier()

    def _fire_inputs(e):
        return [
            pltpu.async_copy(ssrc.at[e], s_src_l, sem_in),
            pltpu.async_copy(sdst.at[e], s_dst_l, sem_in),
            pltpu.async_copy(aeg.at[e].at[pl.ds(base, _EPP)], aeg_l, sem_in),
            pltpu.async_copy(esrc.at[e].at[pl.ds(base, _EPP)], src_l, sem_in),
            pltpu.async_copy(edst.at[e].at[pl.ds(base, _EPP)], dst_l, sem_in),
        ]

    def _drain_inputs():
        pltpu.make_async_copy(ssrc.at[0], s_src_l, sem_in).wait()
        pltpu.make_async_copy(sdst.at[0], s_dst_l, sem_in).wait()
        pltpu.make_async_copy(aeg.at[0].at[pl.ds(0, _EPP)], aeg_l, sem_in).wait()
        pltpu.make_async_copy(esrc.at[0].at[pl.ds(0, _EPP)], src_l, sem_in).wait()
        pltpu.make_async_copy(edst.at[0].at[pl.ds(0, _EPP)], dst_l, sem_in).wait()

    _fire_inputs(jnp.minimum(c * _EPC, 24))

    def _per_edge_type(i, carry):
        e = jnp.minimum(c * _EPC + i, 24)
        st = e // _NT
        p = i % 2
        q = 1 - p

        _drain_inputs()

        @pl.when(i > 0)
        def _():
            e_prev = jnp.minimum(c * _EPC + i - 1, 24)
            pltpu.sync_copy(agg_sh.at[pl.ds(s * _RPS, _RPS)],
                            agg_out.at[e_prev].at[pl.ds(s * _RPS, _RPS)])
        d_z = [
            pltpu.async_copy(zbuf, agg_sh.at[pl.ds(s * _RPS, _RPS)], sem_z),
            pltpu.async_copy(z816, den_sh.at[pl.ds(q * 128 + s * 8, 8)], sem_z),
        ]

        def _zd(r, carry2):
            den_l[r] = zf
            return carry2
        lax.fori_loop(0, 128, _zd, 0)

        def _idx(g, carry2):
            o = g * 16
            src2d[g // 5, pl.ds((g % 5) * 16, 16)] = src_l[pl.ds(o, 16)] + st * _N
            dst2d[g // 5, pl.ds((g % 5) * 16, 16)] = dst_l[pl.ds(o, 16)]
            return carry2
        lax.fori_loop(0, _NB * 5, _idx, 0)
        d_g = [pltpu.async_copy(xflat.at[src2d.at[j]],
                                rows.at[pl.ds(j * _BSZ, _BSZ)], sem_g)
               for j in range(_NB)]

        def _p1(g, carry2):
            o = g * 16
            vs = src_l[pl.ds(o, 16)]
            vd = dst_l[pl.ds(o, 16)]
            a1 = plsc.load_gather(s_src_l, [vs])
            a2 = plsc.load_gather(s_dst_l, [vd])
            al = a1 + a2 + aeg_l[pl.ds(o, 16)]
            al = jnp.where(al >= 0, al, al * 0.2)
            ex = jnp.exp(al)
            ex_l[pl.ds(o, 16)] = ex
            plsc.addupdate_scatter(den_l, [vd // 16, vd % 16], ex)
            return carry2
        lax.fori_loop(0, _EPS // 16, _p1, 0)

        pltpu.sync_copy(den_l, den_sh.at[iota_r.at[p]], add=True)
        for d in d_z:
            d.wait()
        plsc.subcore_barrier()
        pltpu.sync_copy(den_sh.at[pl.ds(p * 128, 128)], den_l)

        def _p2(g, carry2):
            o = g * 16
            vd = dst_l[pl.ds(o, 16)]
            dv = plsc.load_gather(den_l, [vd // 16, vd % 16])
            a_l[pl.ds(o, 16)] = ex_l[pl.ds(o, 16)] / (dv + 1e-16)
            return carry2
        lax.fori_loop(0, _EPS // 16, _p2, 0)

        _fire_inputs(jnp.minimum(c * _EPC + i + 1, 24))

        d_s = []
        for j in range(_NB):
            d_g[j].wait()

            def _scale(g2, carry2):
                i2 = g2 * 2
                ab0 = plsc.load_gather(a_l, [jnp.full((16,), 0, jnp.int32) + i2])
                ab1 = plsc.load_gather(a_l, [jnp.full((16,), 1, jnp.int32) + i2])
                for k in range(8):
                    rows[i2, pl.ds(k * 16, 16)] = rows[i2, pl.ds(k * 16, 16)] * ab0
                for k in range(8):
                    rows[i2 + 1, pl.ds(k * 16, 16)] = (
                        rows[i2 + 1, pl.ds(k * 16, 16)] * ab1)
                return carry2
            lax.fori_loop(j * _BSZ // 2, (j + 1) * _BSZ // 2, _scale, 0)
            d_s.append(pltpu.async_copy(rows.at[pl.ds(j * _BSZ, _BSZ)],
                                        agg_sh.at[dst2d.at[j]], sem_s,
                                        add=True))
        for d in d_s:
            d.wait()
        plsc.subcore_barrier()
        return carry

    lax.fori_loop(0, _EPC, _per_edge_type, 0)

    _drain_inputs()
    e_last = jnp.minimum(c * _EPC + _EPC - 1, 24)
    pltpu.sync_copy(agg_sh.at[pl.ds(s * _RPS, _RPS)],
                    agg_out.at[e_last].at[pl.ds(s * _RPS, _RPS)])


@functools.cache
def _make_sc_edge_aggregate():
    @functools.partial(
        pl.kernel,
        mesh=plsc.VectorSubcoreMesh(core_axis_name="c", subcore_axis_name="s"),
        out_type=jax.ShapeDtypeStruct((_NT * _NT, _NP, _H), jnp.float32),
        compiler_params=pltpu.CompilerParams(needs_layout_passes=False),
        scratch_types=[
            pltpu.VMEM((_N,), jnp.float32),
            pltpu.VMEM((_N,), jnp.float32),
            pltpu.VMEM((_EPP,), jnp.float32),
            pltpu.VMEM((_EPP,), jnp.int32),
            pltpu.VMEM((_EPP,), jnp.int32),
            pltpu.VMEM((_NB, _BSZ), jnp.int32),
            pltpu.VMEM((_NB, _BSZ), jnp.int32),
            pltpu.VMEM((_EPP,), jnp.float32),
            pltpu.VMEM((_EPP,), jnp.float32),
            pltpu.VMEM((128, 16), jnp.float32),
            pltpu.VMEM((_EPP, _H), jnp.float32),
            pltpu.VMEM((_RPS, _H), jnp.float32),
            pltpu.VMEM((8, 16), jnp.float32),
            pltpu.VMEM((2, 128), jnp.int32),
            pltpu.VMEM_SHARED((256, 16), jnp.float32),
            pltpu.VMEM_SHARED((_NP, _H), jnp.float32),
            pltpu.SemaphoreType.DMA,
            pltpu.SemaphoreType.DMA,
            pltpu.SemaphoreType.DMA,
            pltpu.SemaphoreType.DMA,
        ],
    )
    def _sc_edge_aggregate(xflat, ssrc, sdst, aeg, esrc, edst, agg_out, *rest):
        _sc_edge_body(xflat, ssrc, sdst, aeg, esrc, edst, agg_out, *rest)

    return _sc_edge_aggregate


def _gate_block(agg_ref, Wg_ref, bg_ref, x_ref, Wxc_ref, bxc_ref,
                Wcc_ref, bcc_ref, Wat_ref, bat_ref):
    comms = None
    for st in range(_NT):
        o = jnp.dot(agg_ref[st, 0], Wg_ref[st, 0],
                    preferred_element_type=jnp.float32) + bg_ref[st, 0]
        comms = o if comms is None else jnp.maximum(comms, o)
    cc = _lrelu(comms, 0.01)
    xv = x_ref[0]
    left = jnp.dot(xv, Wxc_ref[0], preferred_element_type=jnp.float32) + bxc_ref[0]
    right = jnp.dot(cc, Wcc_ref[0], preferred_element_type=jnp.float32) + bcc_ref[0]
    xt = jnp.concatenate([left, right], axis=1)
    att = jnp.dot(xt, Wat_ref[0], preferred_element_type=jnp.float32) + bat_ref[0]
    return xt + jax.nn.sigmoid(att) * xt


def _a_block(x1, W_ref, b_ref, Wgs_ref, asrc_ref, Wgd_ref, adst_ref,
             eaT_ref, We_ref, ae_ref, xn_ref, S_ref, aeg_ref):
    xn = _lrelu(jnp.dot(x1, W_ref[0], preferred_element_type=jnp.float32)
                + b_ref[0], 0.01)
    xn_ref[0] = xn
    cols = []
    for dt in range(_NT):
        row = asrc_ref[0, dt][None, :]
        cols.append(jnp.sum(Wgs_ref[0, dt] * row, axis=1, keepdims=True))
        erow = ae_ref[0, dt][None, :]
        we = jnp.sum(We_ref[0, dt] * erow, axis=1, keepdims=True)
        aeg_ref[0, dt] = jnp.sum(eaT_ref[0, dt] * we, axis=0, keepdims=True)
    for st in range(_NT):
        drow = adst_ref[st, 0]
        cols.append(jnp.sum(Wgd_ref[st, 0] * drow, axis=1, keepdims=True))
    cols.append(jnp.zeros((_H, _H - 2 * _NT), jnp.float32))
    M = jnp.concatenate(cols, axis=1)
    S_ref[0] = jnp.dot(xn, M, preferred_element_type=jnp.float32)


def _stage_ca_body(agg_ref, Wg_ref, bg_ref, x_ref, Wxc_ref, bxc_ref,
                   Wcc_ref, bcc_ref, Wat_ref, bat_ref,
                   W_ref, b_ref, Wgs_ref, asrc_ref, Wgd_ref, adst_ref,
                   eaT_ref, We_ref, ae_ref,
                   xn_ref, S_ref, aeg_ref):
    x1 = _gate_block(agg_ref, Wg_ref, bg_ref, x_ref, Wxc_ref, bxc_ref,
                     Wcc_ref, bcc_ref, Wat_ref, bat_ref)
    _a_block(x1, W_ref, b_ref, Wgs_ref, asrc_ref, Wgd_ref, adst_ref,
             eaT_ref, We_ref, ae_ref, xn_ref, S_ref, aeg_ref)


def _stage_ca(agg_p, Wg_r, bg_r, xn, Wxc, bxc, Wcc, bcc, Wat, bat,
              W, b, Wg_n, asrc_n, adst_n, eaT_r, We_n, ae_n):
    f32 = jnp.float32
    return pl.pallas_call(
        _stage_ca_body,
        grid=(_NT,),
        in_specs=[
            pl.BlockSpec((_NT, 1, _N, _H), lambda t: (0, t, 0, 0)),
            pl.BlockSpec((_NT, 1, _H, _H), lambda t: (0, t, 0, 0)),
            pl.BlockSpec((_NT, 1, 1, _H), lambda t: (0, t, 0, 0)),
            pl.BlockSpec((1, _N, _H), lambda t: (t, 0, 0)),
            pl.BlockSpec((1, _H, _H // 2), lambda t: (t, 0, 0)),
            pl.BlockSpec((1, 1, _H // 2), lambda t: (t, 0, 0)),
            pl.BlockSpec((1, _H, _H // 2), lambda t: (t, 0, 0)),
            pl.BlockSpec((1, 1, _H // 2), lambda t: (t, 0, 0)),
            pl.BlockSpec((1, _H, _H), lambda t: (t, 0, 0)),
            pl.BlockSpec((1, 1, _H), lambda t: (t, 0, 0)),
            pl.BlockSpec((1, _H, _H), lambda t: (t, 0, 0)),
            pl.BlockSpec((1, 1, _H), lambda t: (t, 0, 0)),
            pl.BlockSpec((1, _NT, _H, _H), lambda t: (t, 0, 0, 0)),
            pl.BlockSpec((1, _NT, _H), lambda t: (t, 0, 0)),
            pl.BlockSpec((_NT, 1, _H, _H), lambda t: (0, t, 0, 0)),
            pl.BlockSpec((_NT, 1, 1, _H), lambda t: (0, t, 0, 0)),
            pl.BlockSpec((1, _NT, 4, _E), lambda t: (t, 0, 0, 0)),
            pl.BlockSpec((1, _NT, 4, _H), lambda t: (t, 0, 0, 0)),
            pl.BlockSpec((1, _NT, _H), lambda t: (t, 0, 0)),
        ],
        out_specs=[
            pl.BlockSpec((1, _N, _H), lambda t: (t, 0, 0)),
            pl.BlockSpec((1, _N, _H), lambda t: (t, 0, 0)),
            pl.BlockSpec((1, _NT, 1, _E), lambda t: (t, 0, 0, 0)),
        ],
        out_shape=[
            jax.ShapeDtypeStruct((_NT, _N, _H), f32),
            jax.ShapeDtypeStruct((_NT, _N, _H), f32),
            jax.ShapeDtypeStruct((_NT, _NT, 1, _E), f32),
        ],
    )(agg_p, Wg_r, bg_r.reshape(_NT, _NT, 1, _H), xn,
      Wxc, bxc.reshape(_NT, 1, _H // 2), Wcc, bcc.reshape(_NT, 1, _H // 2),
      Wat, bat.reshape(_NT, 1, _H),
      W, b.reshape(_NT, 1, _H), Wg_n, asrc_n, Wg_n,
      adst_n.reshape(_NT, _NT, 1, _H), eaT_r, We_n, ae_n)


def _stage_cd_body(agg_ref, Wg_ref, bg_ref, x_ref, Wxc_ref, bxc_ref,
                   Wcc_ref, bcc_ref, Wat_ref, bat_ref,
                   W2_ref, b2_ref, W3_ref, b3_ref, bb_ref, wout_ref,
                   bout_ref, out_ref, acc_ref, cnt_ref):
    t = pl.program_id(0)
    x1 = _gate_block(agg_ref, Wg_ref, bg_ref, x_ref, Wxc_ref, bxc_ref,
                     Wcc_ref, bcc_ref, Wat_ref, bat_ref)
    y = _lrelu(jnp.dot(x1, W2_ref[0], preferred_element_type=jnp.float32)
               + b2_ref[0], 0.01)
    y = _lrelu(jnp.dot(y, W3_ref[0], preferred_element_type=jnp.float32)
               + b3_ref[0], 0.01)
    bb = bb_ref[0, 0]
    iota = lax.broadcasted_iota(jnp.int32, (_N, _NG), 1)
    oh = (iota == bb[:, None]).astype(jnp.float32)
    part = lax.dot_general(oh, y, (((0,), (0,)), ((), ())),
                           preferred_element_type=jnp.float32)
    cpart = lax.dot_general(oh, jnp.ones((_N, 1), jnp.float32),
                            (((0,), (0,)), ((), ())),
                            preferred_element_type=jnp.float32)

    @pl.when(t == 0)
    def _():
        acc_ref[...] = part
        cnt_ref[...] = cpart

    @pl.when(t > 0)
    def _():
        acc_ref[...] += part
        cnt_ref[...] += cpart

    @pl.when(t == _NT - 1)
    def _():
        pooled = acc_ref[...] / jnp.maximum(cnt_ref[...], 1.0)
        out_ref[...] = jax.nn.sigmoid(
            jnp.dot(pooled, wout_ref[...], preferred_element_type=jnp.float32)
            + bout_ref[0])


def _stage_cd(agg_p, Wg_r, bg_r, xn, Wxc, bxc, Wcc, bcc, Wat, bat,
              W2, b2, W3, b3, batch_ids, W_out, b_out):
    f32 = jnp.float32
    return pl.pallas_call(
        _stage_cd_body,
        grid=(_NT,),
        in_specs=[
            pl.BlockSpec((_NT, 1, _N, _H), lambda t: (0, t, 0, 0)),
            pl.BlockSpec((_NT, 1, _H, _H), lambda t: (0, t, 0, 0)),
            pl.BlockSpec((_NT, 1, 1, _H), lambda t: (0, t, 0, 0)),
            pl.BlockSpec((1, _N, _H), lambda t: (t, 0, 0)),
            pl.BlockSpec((1, _H, _H // 2), lambda t: (t, 0, 0)),
            pl.BlockSpec((1, 1, _H // 2), lambda t: (t, 0, 0)),
            pl.BlockSpec((1, _H, _H // 2), lambda t: (t, 0, 0)),
            pl.BlockSpec((1, 1, _H // 2), lambda t: (t, 0, 0)),
            pl.BlockSpec((1, _H, _H), lambda t: (t, 0, 0)),
            pl.BlockSpec((1, 1, _H), lambda t: (t, 0, 0)),
            pl.BlockSpec((1, _H, _H), lambda t: (t, 0, 0)),
            pl.BlockSpec((1, 1, _H), lambda t: (t, 0, 0)),
            pl.BlockSpec((1, _H, _H), lambda t: (t, 0, 0)),
            pl.BlockSpec((1, 1, _H), lambda t: (t, 0, 0)),
            pl.BlockSpec((1, 1, _N), lambda t: (t, 0, 0)),
            pl.BlockSpec((_H, 1), lambda t: (0, 0)),
            pl.BlockSpec((1,), lambda t: (0,)),
        ],
        out_specs=pl.BlockSpec((_NG, 1), lambda t: (0, 0)),
        out_shape=jax.ShapeDtypeStruct((_NG, 1), f32),
        scratch_shapes=[
            pltpu.VMEM((_NG, _H), f32),
            pltpu.VMEM((_NG, 1), f32),
        ],
    )(agg_p, Wg_r, bg_r.reshape(_NT, _NT, 1, _H), xn,
      Wxc, bxc.reshape(_NT, 1, _H // 2), Wcc, bcc.reshape(_NT, 1, _H // 2),
      Wat, bat.reshape(_NT, 1, _H),
      W2, b2.reshape(_NT, 1, _H), W3, b3.reshape(_NT, 1, _H),
      batch_ids.reshape(_NT, 1, _N), W_out, b_out)


def kernel(x, edge_index, edge_attr, batch_ids, W_sl0, b_sl0, W_sl, b_sl,
           W_gat, a_src, a_dst, W_edge, a_edge, b_gat, W_xc, b_xc, W_cc,
           b_cc, W_at, b_at, W_out, b_out):
    def _pad_edges(arr, cval):
        a3 = arr.reshape(_NT * _NT, _NSUB, _E // _NSUB)
        a3 = jnp.pad(a3, ((0, 0), (0, 0), (0, _EPP - _E // _NSUB)),
                     constant_values=cval)
        return a3.reshape(_NT * _NT, _EP)

    esrc = _pad_edges(edge_index[:, 0, :], 0)
    edst = _pad_edges(edge_index[:, 1, :], _N)
    eaT_r = edge_attr.transpose(0, 2, 1).reshape(_NT, _NT, 4, _E)

    Wg_r = [W_gat[L].reshape(_NT, _NT, _H, _H) for L in range(3)]
    asrc_r = [a_src[L].reshape(_NT, _NT, _H) for L in range(3)]
    adst_r = [a_dst[L].reshape(_NT, _NT, _H) for L in range(3)]
    We_r = [W_edge[L].reshape(_NT, _NT, 4, _H) for L in range(3)]
    ae_r = [a_edge[L].reshape(_NT, _NT, _H) for L in range(3)]
    bg_r = [b_gat[L].reshape(_NT, _NT, _H) for L in range(3)]

    def _run_sc(xn, S, aeg4):
        ssrc = S[:, :, :_NT].transpose(0, 2, 1).reshape(_NT * _NT, _N)
        sdst = S[:, :, _NT:2 * _NT].transpose(2, 0, 1).reshape(_NT * _NT, _N)
        aeg_p = _pad_edges(aeg4.reshape(_NT * _NT, _E), 0.0)
        agg = _make_sc_edge_aggregate()(
            xn.reshape(_NT * _N, _H), ssrc, sdst, aeg_p, esrc, edst)
        return agg.reshape(_NT, _NT, _NP, _H)

    xn, S, aeg4 = _stage_a(x, W_sl0, b_sl0, Wg_r[0], asrc_r[0], adst_r[0],
                           eaT_r, We_r[0], ae_r[0])
    agg_p = _run_sc(xn, S, aeg4)
    for L in range(2):
        xn, S, aeg4 = _stage_ca(
            agg_p, Wg_r[L], bg_r[L], xn,
            W_xc[L], b_xc[L], W_cc[L], b_cc[L], W_at[L], b_at[L],
            W_sl[L], b_sl[L], Wg_r[L + 1], asrc_r[L + 1], adst_r[L + 1],
            eaT_r, We_r[L + 1], ae_r[L + 1])
        agg_p = _run_sc(xn, S, aeg4)
    return _stage_cd(agg_p, Wg_r[2], bg_r[2], xn,
                     W_xc[2], b_xc[2], W_cc[2], b_cc[2], W_at[2], b_at[2],
                     W_sl[2], b_sl[2], W_sl[3], b_sl[3],
                     batch_ids, W_out, b_out)


# speedup vs baseline: 6.8686x; 1.0152x over previous
import functools

import jax
import jax.numpy as jnp
from jax import lax
from jax.experimental import pallas as pl
from jax.experimental.pallas import tpu as pltpu
from jax.experimental.pallas import tpu_sc as plsc

_NT = 5
_NG = 256
_N = 2000
_E = 6400
_H = 128

_NSUB = 16
_EPP = 512
_EPS = 400
_BSZ = 80
_NB = _EPS // _BSZ
_EP = _EPP * _NSUB
_NP = 2048
_RPS = _NP // _NSUB
_EPC = 13


def _lrelu(x, slope):
    return jnp.where(x >= 0, x, x * slope)


def _stage_a_body(x_ref, W_ref, b_ref, Wgs_ref, asrc_ref, Wgd_ref, adst_ref,
                  eaT_ref, We_ref, ae_ref,
                  xn_ref, S_ref, aeg_ref):
    xn = _lrelu(jnp.dot(x_ref[0], W_ref[0], preferred_element_type=jnp.float32)
                + b_ref[0], 0.01)
    xn_ref[0] = xn
    cols = []
    for dt in range(_NT):
        row = asrc_ref[0, dt][None, :]
        cols.append(jnp.sum(Wgs_ref[0, dt] * row, axis=1, keepdims=True))
        erow = ae_ref[0, dt][None, :]
        we = jnp.sum(We_ref[0, dt] * erow, axis=1, keepdims=True)
        aeg_ref[0, dt] = jnp.sum(eaT_ref[0, dt] * we, axis=0, keepdims=True)
    for st in range(_NT):
        drow = adst_ref[st, 0]
        cols.append(jnp.sum(Wgd_ref[st, 0] * drow, axis=1, keepdims=True))
    cols.append(jnp.zeros((_H, _H - 2 * _NT), jnp.float32))
    M = jnp.concatenate(cols, axis=1)
    S_ref[0] = jnp.dot(xn, M, preferred_element_type=jnp.float32)


def _stage_a(x, W, b, Wg_r, asrc_r, adst_r, eaT_r, We_r, ae_r):
    din = x.shape[-1]
    f32 = jnp.float32
    return pl.pallas_call(
        _stage_a_body,
        grid=(_NT,),
        in_specs=[
            pl.BlockSpec((1, _N, din), lambda t: (t, 0, 0)),
            pl.BlockSpec((1, din, _H), lambda t: (t, 0, 0)),
            pl.BlockSpec((1, 1, _H), lambda t: (t, 0, 0)),
            pl.BlockSpec((1, _NT, _H, _H), lambda t: (t, 0, 0, 0)),
            pl.BlockSpec((1, _NT, _H), lambda t: (t, 0, 0)),
            pl.BlockSpec((_NT, 1, _H, _H), lambda t: (0, t, 0, 0)),
            pl.BlockSpec((_NT, 1, 1, _H), lambda t: (0, t, 0, 0)),
            pl.BlockSpec((1, _NT, 4, _E), lambda t: (t, 0, 0, 0)),
            pl.BlockSpec((1, _NT, 4, _H), lambda t: (t, 0, 0, 0)),
            pl.BlockSpec((1, _NT, _H), lambda t: (t, 0, 0)),
        ],
        out_specs=[
            pl.BlockSpec((1, _N, _H), lambda t: (t, 0, 0)),
            pl.BlockSpec((1, _N, _H), lambda t: (t, 0, 0)),
            pl.BlockSpec((1, _NT, 1, _E), lambda t: (t, 0, 0, 0)),
        ],
        out_shape=[
            jax.ShapeDtypeStruct((_NT, _N, _H), f32),
            jax.ShapeDtypeStruct((_NT, _N, _H), f32),
            jax.ShapeDtypeStruct((_NT, _NT, 1, _E), f32),
        ],
    )(x, W, b.reshape(_NT, 1, _H), Wg_r, asrc_r, Wg_r,
      adst_r.reshape(_NT, _NT, 1, _H), eaT_r, We_r, ae_r)


def _sc_edge_body(xflat, ssrc, sdst, aeg, esrc, edst, agg_out,
                  s_src_l, s_dst_l, aeg_l, src_l, dst_l, src2d, dst2d,
                  ex_l, a_l, den_l, rows, zbuf, z816, iota_r, den_sh,
                  agg_sh, sem_in, sem_z, sem_g, sem_s):
    c = lax.axis_index("c")
    s = lax.axis_index("s")
    base = s * _EPP
    zf = jnp.zeros((16,), jnp.float32)

    def _zb(i, carry):
        r = i // 8
        k = i % 8
        zbuf[r, pl.ds(k * 16, 16)] = zf
        return carry
    lax.fori_loop(0, _RPS * 8, _zb, 0)
    for r in range(8):
        z816[r, :] = zf
    for p2 in range(2):
        for k in range(8):
            iota_r[p2, pl.ds(k * 16, 16)] = (lax.iota(jnp.int32, 16)
                                             + k * 16 + p2 * 128)

    pltpu.sync_copy(zbuf, agg_sh.at[pl.ds(s * _RPS, _RPS)])
    pltpu.sync_copy(z816, den_sh.at[pl.ds(s * 8, 8)])
    plsc.subcore_barrier()

    def _fire_inputs(e):
        return [
            pltpu.async_copy(ssrc.at[e], s_src_l, sem_in),
            pltpu.async_copy(sdst.at[e], s_dst_l, sem_in),
            pltpu.async_copy(aeg.at[e].at[pl.ds(base, _EPP)], aeg_l, sem_in),
            pltpu.async_copy(esrc.at[e].at[pl.ds(base, _EPP)], src_l, sem_in),
            pltpu.async_copy(edst.at[e].at[pl.ds(base, _EPP)], dst_l, sem_in),
        ]

    def _drain_inputs():
        pltpu.make_async_copy(ssrc.at[0], s_src_l, sem_in).wait()
        pltpu.make_async_copy(sdst.at[0], s_dst_l, sem_in).wait()
        pltpu.make_async_copy(aeg.at[0].at[pl.ds(0, _EPP)], aeg_l, sem_in).wait()
        pltpu.make_async_copy(esrc.at[0].at[pl.ds(0, _EPP)], src_l, sem_in).wait()
        pltpu.make_async_copy(edst.at[0].at[pl.ds(0, _EPP)], dst_l, sem_in).wait()

    _fire_inputs(jnp.minimum(c * _EPC, 24))

    def _per_edge_type(i, carry):
        e = jnp.minimum(c * _EPC + i, 24)
        st = e // _NT
        p = i % 2
        q = 1 - p

        _drain_inputs()

        def _idx(g, carry2):
            o = g * 16
            src2d[g // 5, pl.ds((g % 5) * 16, 16)] = src_l[pl.ds(o, 16)] + st * _N
            dst2d[g // 5, pl.ds((g % 5) * 16, 16)] = dst_l[pl.ds(o, 16)]
            return carry2
        lax.fori_loop(0, _NB * 5, _idx, 0)
        d_g = [pltpu.async_copy(xflat.at[src2d.at[j]],
                                rows.at[pl.ds(j * _BSZ, _BSZ)], sem_g)
               for j in range(_NB)]

        @pl.when(i > 0)
        def _():
            e_prev = jnp.minimum(c * _EPC + i - 1, 24)
            pltpu.sync_copy(agg_sh.at[pl.ds(s * _RPS, _RPS)],
                            agg_out.at[e_prev].at[pl.ds(s * _RPS, _RPS)])
        d_z = [
            pltpu.async_copy(zbuf, agg_sh.at[pl.ds(s * _RPS, _RPS)], sem_z),
            pltpu.async_copy(z816, den_sh.at[pl.ds(q * 128 + s * 8, 8)], sem_z),
        ]

        def _zd(r, carry2):
            den_l[r] = zf
            return carry2
        lax.fori_loop(0, 128, _zd, 0)

        def _p1(g, carry2):
            o = g * 16
            vs = src_l[pl.ds(o, 16)]
            vd = dst_l[pl.ds(o, 16)]
            a1 = plsc.load_gather(s_src_l, [vs])
            a2 = plsc.load_gather(s_dst_l, [vd])
            al = a1 + a2 + aeg_l[pl.ds(o, 16)]
            al = jnp.where(al >= 0, al, al * 0.2)
            ex = jnp.exp(al)
            ex_l[pl.ds(o, 16)] = ex
            plsc.addupdate_scatter(den_l, [vd // 16, vd % 16], ex)
            return carry2
        lax.fori_loop(0, _EPS // 16, _p1, 0)

        pltpu.sync_copy(den_l, den_sh.at[iota_r.at[p]], add=True)
        for d in d_z:
            d.wait()
        plsc.subcore_barrier()
        pltpu.sync_copy(den_sh.at[pl.ds(p * 128, 128)], den_l)

        def _p2(g, carry2):
            o = g * 16
            vd = dst_l[pl.ds(o, 16)]
            dv = plsc.load_gather(den_l, [vd // 16, vd % 16])
            a_l[pl.ds(o, 16)] = ex_l[pl.ds(o, 16)] / (dv + 1e-16)
            return carry2
        lax.fori_loop(0, _EPS // 16, _p2, 0)

        _fire_inputs(jnp.minimum(c * _EPC + i + 1, 24))

        d_s = []
        for j in range(_NB):
            d_g[j].wait()

            def _scale(g2, carry2):
                i2 = g2 * 2
                ab0 = plsc.load_gather(a_l, [jnp.full((16,), 0, jnp.int32) + i2])
                ab1 = plsc.load_gather(a_l, [jnp.full((16,), 1, jnp.int32) + i2])
                for k in range(8):
                    rows[i2, pl.ds(k * 16, 16)] = rows[i2, pl.ds(k * 16, 16)] * ab0
                for k in range(8):
                    rows[i2 + 1, pl.ds(k * 16, 16)] = (
                        rows[i2 + 1, pl.ds(k * 16, 16)] * ab1)
                return carry2
            lax.fori_loop(j * _BSZ // 2, (j + 1) * _BSZ // 2, _scale, 0)
            d_s.append(pltpu.async_copy(rows.at[pl.ds(j * _BSZ, _BSZ)],
                                        agg_sh.at[dst2d.at[j]], sem_s,
                                        add=True))
        for d in d_s:
            d.wait()
        plsc.subcore_barrier()
        return carry

    lax.fori_loop(0, _EPC, _per_edge_type, 0)

    _drain_inputs()
    e_last = jnp.minimum(c * _EPC + _EPC - 1, 24)
    pltpu.sync_copy(agg_sh.at[pl.ds(s * _RPS, _RPS)],
                    agg_out.at[e_last].at[pl.ds(s * _RPS, _RPS)])


@functools.cache
def _make_sc_edge_aggregate():
    @functools.partial(
        pl.kernel,
        mesh=plsc.VectorSubcoreMesh(core_axis_name="c", subcore_axis_name="s"),
        out_type=jax.ShapeDtypeStruct((_NT * _NT, _NP, _H), jnp.float32),
        compiler_params=pltpu.CompilerParams(needs_layout_passes=False),
        scratch_types=[
            pltpu.VMEM((_N,), jnp.float32),
            pltpu.VMEM((_N,), jnp.float32),
            pltpu.VMEM((_EPP,), jnp.float32),
            pltpu.VMEM((_EPP,), jnp.int32),
            pltpu.VMEM((_EPP,), jnp.int32),
            pltpu.VMEM((_NB, _BSZ), jnp.int32),
            pltpu.VMEM((_NB, _BSZ), jnp.int32),
            pltpu.VMEM((_EPP,), jnp.float32),
            pltpu.VMEM((_EPP,), jnp.float32),
            pltpu.VMEM((128, 16), jnp.float32),
            pltpu.VMEM((_EPP, _H), jnp.float32),
            pltpu.VMEM((_RPS, _H), jnp.float32),
            pltpu.VMEM((8, 16), jnp.float32),
            pltpu.VMEM((2, 128), jnp.int32),
            pltpu.VMEM_SHARED((256, 16), jnp.float32),
            pltpu.VMEM_SHARED((_NP, _H), jnp.float32),
            pltpu.SemaphoreType.DMA,
            pltpu.SemaphoreType.DMA,
            pltpu.SemaphoreType.DMA,
            pltpu.SemaphoreType.DMA,
        ],
    )
    def _sc_edge_aggregate(xflat, ssrc, sdst, aeg, esrc, edst, agg_out, *rest):
        _sc_edge_body(xflat, ssrc, sdst, aeg, esrc, edst, agg_out, *rest)

    return _sc_edge_aggregate


def _gate_block(agg_ref, Wg_ref, bg_ref, x_ref, Wxc_ref, bxc_ref,
                Wcc_ref, bcc_ref, Wat_ref, bat_ref):
    comms = None
    for st in range(_NT):
        o = jnp.dot(agg_ref[st, 0], Wg_ref[st, 0],
                    preferred_element_type=jnp.float32) + bg_ref[st, 0]
        comms = o if comms is None else jnp.maximum(comms, o)
    cc = _lrelu(comms, 0.01)
    xv = x_ref[0]
    left = jnp.dot(xv, Wxc_ref[0], preferred_element_type=jnp.float32) + bxc_ref[0]
    right = jnp.dot(cc, Wcc_ref[0], preferred_element_type=jnp.float32) + bcc_ref[0]
    xt = jnp.concatenate([left, right], axis=1)
    att = jnp.dot(xt, Wat_ref[0], preferred_element_type=jnp.float32) + bat_ref[0]
    return xt + jax.nn.sigmoid(att) * xt


def _a_block(x1, W_ref, b_ref, Wgs_ref, asrc_ref, Wgd_ref, adst_ref,
             eaT_ref, We_ref, ae_ref, xn_ref, S_ref, aeg_ref):
    xn = _lrelu(jnp.dot(x1, W_ref[0], preferred_element_type=jnp.float32)
                + b_ref[0], 0.01)
    xn_ref[0] = xn
    cols = []
    for dt in range(_NT):
        row = asrc_ref[0, dt][None, :]
        cols.append(jnp.sum(Wgs_ref[0, dt] * row, axis=1, keepdims=True))
        erow = ae_ref[0, dt][None, :]
        we = jnp.sum(We_ref[0, dt] * erow, axis=1, keepdims=True)
        aeg_ref[0, dt] = jnp.sum(eaT_ref[0, dt] * we, axis=0, keepdims=True)
    for st in range(_NT):
        drow = adst_ref[st, 0]
        cols.append(jnp.sum(Wgd_ref[st, 0] * drow, axis=1, keepdims=True))
    cols.append(jnp.zeros((_H, _H - 2 * _NT), jnp.float32))
    M = jnp.concatenate(cols, axis=1)
    S_ref[0] = jnp.dot(xn, M, preferred_element_type=jnp.float32)


def _stage_ca_body(agg_ref, Wg_ref, bg_ref, x_ref, Wxc_ref, bxc_ref,
                   Wcc_ref, bcc_ref, Wat_ref, bat_ref,
                   W_ref, b_ref, Wgs_ref, asrc_ref, Wgd_ref, adst_ref,
                   eaT_ref, We_ref, ae_ref,
                   xn_ref, S_ref, aeg_ref):
    x1 = _gate_block(agg_ref, Wg_ref, bg_ref, x_ref, Wxc_ref, bxc_ref,
                     Wcc_ref, bcc_ref, Wat_ref, bat_ref)
    _a_block(x1, W_ref, b_ref, Wgs_ref, asrc_ref, Wgd_ref, adst_ref,
             eaT_ref, We_ref, ae_ref, xn_ref, S_ref, aeg_ref)


def _stage_ca(agg_p, Wg_r, bg_r, xn, Wxc, bxc, Wcc, bcc, Wat, bat,
              W, b, Wg_n, asrc_n, adst_n, eaT_r, We_n, ae_n):
    f32 = jnp.float32
    return pl.pallas_call(
        _stage_ca_body,
        grid=(_NT,),
        in_specs=[
            pl.BlockSpec((_NT, 1, _N, _H), lambda t: (0, t, 0, 0)),
            pl.BlockSpec((_NT, 1, _H, _H), lambda t: (0, t, 0, 0)),
            pl.BlockSpec((_NT, 1, 1, _H), lambda t: (0, t, 0, 0)),
            pl.BlockSpec((1, _N, _H), lambda t: (t, 0, 0)),
            pl.BlockSpec((1, _H, _H // 2), lambda t: (t, 0, 0)),
            pl.BlockSpec((1, 1, _H // 2), lambda t: (t, 0, 0)),
            pl.BlockSpec((1, _H, _H // 2), lambda t: (t, 0, 0)),
            pl.BlockSpec((1, 1, _H // 2), lambda t: (t, 0, 0)),
            pl.BlockSpec((1, _H, _H), lambda t: (t, 0, 0)),
            pl.BlockSpec((1, 1, _H), lambda t: (t, 0, 0)),
            pl.BlockSpec((1, _H, _H), lambda t: (t, 0, 0)),
            pl.BlockSpec((1, 1, _H), lambda t: (t, 0, 0)),
            pl.BlockSpec((1, _NT, _H, _H), lambda t: (t, 0, 0, 0)),
            pl.BlockSpec((1, _NT, _H), lambda t: (t, 0, 0)),
            pl.BlockSpec((_NT, 1, _H, _H), lambda t: (0, t, 0, 0)),
            pl.BlockSpec((_NT, 1, 1, _H), lambda t: (0, t, 0, 0)),
            pl.BlockSpec((1, _NT, 4, _E), lambda t: (t, 0, 0, 0)),
            pl.BlockSpec((1, _NT, 4, _H), lambda t: (t, 0, 0, 0)),
            pl.BlockSpec((1, _NT, _H), lambda t: (t, 0, 0)),
        ],
        out_specs=[
            pl.BlockSpec((1, _N, _H), lambda t: (t, 0, 0)),
            pl.BlockSpec((1, _N, _H), lambda t: (t, 0, 0)),
            pl.BlockSpec((1, _NT, 1, _E), lambda t: (t, 0, 0, 0)),
        ],
        out_shape=[
            jax.ShapeDtypeStruct((_NT, _N, _H), f32),
            jax.ShapeDtypeStruct((_NT, _N, _H), f32),
            jax.ShapeDtypeStruct((_NT, _NT, 1, _E), f32),
        ],
    )(agg_p, Wg_r, bg_r.reshape(_NT, _NT, 1, _H), xn,
      Wxc, bxc.reshape(_NT, 1, _H // 2), Wcc, bcc.reshape(_NT, 1, _H // 2),
      Wat, bat.reshape(_NT, 1, _H),
      W, b.reshape(_NT, 1, _H), Wg_n, asrc_n, Wg_n,
      adst_n.reshape(_NT, _NT, 1, _H), eaT_r, We_n, ae_n)


def _stage_cd_body(agg_ref, Wg_ref, bg_ref, x_ref, Wxc_ref, bxc_ref,
                   Wcc_ref, bcc_ref, Wat_ref, bat_ref,
                   W2_ref, b2_ref, W3_ref, b3_ref, bb_ref, wout_ref,
                   bout_ref, out_ref, acc_ref, cnt_ref):
    t = pl.program_id(0)
    x1 = _gate_block(agg_ref, Wg_ref, bg_ref, x_ref, Wxc_ref, bxc_ref,
                     Wcc_ref, bcc_ref, Wat_ref, bat_ref)
    y = _lrelu(jnp.dot(x1, W2_ref[0], preferred_element_type=jnp.float32)
               + b2_ref[0], 0.01)
    y = _lrelu(jnp.dot(y, W3_ref[0], preferred_element_type=jnp.float32)
               + b3_ref[0], 0.01)
    bb = bb_ref[0, 0]
    iota = lax.broadcasted_iota(jnp.int32, (_N, _NG), 1)
    oh = (iota == bb[:, None]).astype(jnp.float32)
    part = lax.dot_general(oh, y, (((0,), (0,)), ((), ())),
                           preferred_element_type=jnp.float32)
    cpart = lax.dot_general(oh, jnp.ones((_N, 1), jnp.float32),
                            (((0,), (0,)), ((), ())),
                            preferred_element_type=jnp.float32)

    @pl.when(t == 0)
    def _():
        acc_ref[...] = part
        cnt_ref[...] = cpart

    @pl.when(t > 0)
    def _():
        acc_ref[...] += part
        cnt_ref[...] += cpart

    @pl.when(t == _NT - 1)
    def _():
        pooled = acc_ref[...] / jnp.maximum(cnt_ref[...], 1.0)
        out_ref[...] = jax.nn.sigmoid(
            jnp.dot(pooled, wout_ref[...], preferred_element_type=jnp.float32)
            + bout_ref[0])


def _stage_cd(agg_p, Wg_r, bg_r, xn, Wxc, bxc, Wcc, bcc, Wat, bat,
              W2, b2, W3, b3, batch_ids, W_out, b_out):
    f32 = jnp.float32
    return pl.pallas_call(
        _stage_cd_body,
        grid=(_NT,),
        in_specs=[
            pl.BlockSpec((_NT, 1, _N, _H), lambda t: (0, t, 0, 0)),
            pl.BlockSpec((_NT, 1, _H, _H), lambda t: (0, t, 0, 0)),
            pl.BlockSpec((_NT, 1, 1, _H), lambda t: (0, t, 0, 0)),
            pl.BlockSpec((1, _N, _H), lambda t: (t, 0, 0)),
            pl.BlockSpec((1, _H, _H // 2), lambda t: (t, 0, 0)),
            pl.BlockSpec((1, 1, _H // 2), lambda t: (t, 0, 0)),
            pl.BlockSpec((1, _H, _H // 2), lambda t: (t, 0, 0)),
            pl.BlockSpec((1, 1, _H // 2), lambda t: (t, 0, 0)),
            pl.BlockSpec((1, _H, _H), lambda t: (t, 0, 0)),
            pl.BlockSpec((1, 1, _H), lambda t: (t, 0, 0)),
            pl.BlockSpec((1, _H, _H), lambda t: (t, 0, 0)),
            pl.BlockSpec((1, 1, _H), lambda t: (t, 0, 0)),
            pl.BlockSpec((1, _H, _H), lambda t: (t, 0, 0)),
            pl.BlockSpec((1, 1, _H), lambda t: (t, 0, 0)),
            pl.BlockSpec((1, 1, _N), lambda t: (t, 0, 0)),
            pl.BlockSpec((_H, 1), lambda t: (0, 0)),
            pl.BlockSpec((1,), lambda t: (0,)),
        ],
        out_specs=pl.BlockSpec((_NG, 1), lambda t: (0, 0)),
        out_shape=jax.ShapeDtypeStruct((_NG, 1), f32),
        scratch_shapes=[
            pltpu.VMEM((_NG, _H), f32),
            pltpu.VMEM((_NG, 1), f32),
        ],
    )(agg_p, Wg_r, bg_r.reshape(_NT, _NT, 1, _H), xn,
      Wxc, bxc.reshape(_NT, 1, _H // 2), Wcc, bcc.reshape(_NT, 1, _H // 2),
      Wat, bat.reshape(_NT, 1, _H),
      W2, b2.reshape(_NT, 1, _H), W3, b3.reshape(_NT, 1, _H),
      batch_ids.reshape(_NT, 1, _N), W_out, b_out)


def kernel(x, edge_index, edge_attr, batch_ids, W_sl0, b_sl0, W_sl, b_sl,
           W_gat, a_src, a_dst, W_edge, a_edge, b_gat, W_xc, b_xc, W_cc,
           b_cc, W_at, b_at, W_out, b_out):
    def _pad_edges(arr, cval):
        a3 = arr.reshape(_NT * _NT, _NSUB, _E // _NSUB)
        a3 = jnp.pad(a3, ((0, 0), (0, 0), (0, _EPP - _E // _NSUB)),
                     constant_values=cval)
        return a3.reshape(_NT * _NT, _EP)

    esrc = _pad_edges(edge_index[:, 0, :], 0)
    edst = _pad_edges(edge_index[:, 1, :], _N)
    eaT_r = edge_attr.transpose(0, 2, 1).reshape(_NT, _NT, 4, _E)

    Wg_r = [W_gat[L].reshape(_NT, _NT, _H, _H) for L in range(3)]
    asrc_r = [a_src[L].reshape(_NT, _NT, _H) for L in range(3)]
    adst_r = [a_dst[L].reshape(_NT, _NT, _H) for L in range(3)]
    We_r = [W_edge[L].reshape(_NT, _NT, 4, _H) for L in range(3)]
    ae_r = [a_edge[L].reshape(_NT, _NT, _H) for L in range(3)]
    bg_r = [b_gat[L].reshape(_NT, _NT, _H) for L in range(3)]

    def _run_sc(xn, S, aeg4):
        ssrc = S[:, :, :_NT].transpose(0, 2, 1).reshape(_NT * _NT, _N)
        sdst = S[:, :, _NT:2 * _NT].transpose(2, 0, 1).reshape(_NT * _NT, _N)
        aeg_p = _pad_edges(aeg4.reshape(_NT * _NT, _E), 0.0)
        agg = _make_sc_edge_aggregate()(
            xn.reshape(_NT * _N, _H), ssrc, sdst, aeg_p, esrc, edst)
        return agg.reshape(_NT, _NT, _NP, _H)

    xn, S, aeg4 = _stage_a(x, W_sl0, b_sl0, Wg_r[0], asrc_r[0], adst_r[0],
                           eaT_r, We_r[0], ae_r[0])
    agg_p = _run_sc(xn, S, aeg4)
    for L in range(2):
        xn, S, aeg4 = _stage_ca(
            agg_p, Wg_r[L], bg_r[L], xn,
            W_xc[L], b_xc[L], W_cc[L], b_cc[L], W_at[L], b_at[L],
            W_sl[L], b_sl[L], Wg_r[L + 1], asrc_r[L + 1], adst_r[L + 1],
            eaT_r, We_r[L + 1], ae_r[L + 1])
        agg_p = _run_sc(xn, S, aeg4)
    return _stage_cd(agg_p, Wg_r[2], bg_r[2], xn,
                     W_xc[2], b_xc[2], W_cc[2], b_cc[2], W_at[2], b_at[2],
                     W_sl[2], b_sl[2], W_sl[3], b_sl[3],
                     batch_ids, W_out, b_out)
